# baseline (speedup 1.0000x reference)
import functools

import jax
import jax.numpy as jnp
from jax import lax
from jax.experimental import pallas as pl
from jax.experimental.pallas import tpu as pltpu
from jax.experimental.pallas import tpu_sc as plsc

T, D, F, E, K = 2048, 1024, 2048, 8, 2
LANES = 128
TB = 256
NTB = T // TB
BT = 256
NBR = (T * K + E * BT) // BT
GR = NBR * BT
NC, NS = 2, 16
NW = NC * NS
TPW = T // NW
DCH = 16

_HI = lax.Precision.HIGHEST


def _fiota(shape, dim):
    return lax.broadcasted_iota(jnp.int32, shape, dim).astype(jnp.float32)


def _route_shared_body(x_ref, gwt_ref, sw1_ref, sb1_ref, sw3_ref, sb3_ref,
                       sw2_ref, sb2_ref,
                       z_ref, a1_ref, a2_ref, w1o_ref, w2o_ref, rank_ref,
                       cnt_ref, carry):
    pid = pl.program_id(0)

    @pl.when(pid == 0)
    def _():
        carry[...] = jnp.zeros_like(carry)

    x = x_ref[...]
    xb = x.astype(jnp.bfloat16)
    h1 = jnp.dot(xb, sw1_ref[...], preferred_element_type=jnp.float32)
    h1 = h1 + sb1_ref[...]
    h3 = jnp.dot(xb, sw3_ref[...], preferred_element_type=jnp.float32)
    h3 = h3 + sb3_ref[...]
    hh = (h1 * jax.nn.sigmoid(h1)) * h3
    z = jnp.dot(hh.astype(jnp.bfloat16), sw2_ref[...],
                preferred_element_type=jnp.float32)
    z_ref[...] = z + sb2_ref[...]

    s = jnp.dot(x, gwt_ref[...], preferred_element_type=jnp.float32)
    lane = _fiota((TB, LANES), 1)
    valid = lane < E
    sm = jnp.where(valid, s, -jnp.inf)
    mx = jnp.max(sm, axis=1, keepdims=True)
    ex = jnp.exp(sm - mx)
    p = ex / jnp.sum(ex, axis=1, keepdims=True)
    m1 = jnp.max(p, axis=1, keepdims=True)
    is1 = jnp.logical_and(p == m1, valid)
    a1 = jnp.min(jnp.where(is1, lane, float(LANES)), axis=1, keepdims=True)
    not1 = lane != a1
    p2 = jnp.where(jnp.logical_and(not1, valid), p, -1.0)
    m2 = jnp.max(p2, axis=1, keepdims=True)
    is2 = jnp.logical_and(p2 == m2, jnp.logical_and(valid, not1))
    a2 = jnp.min(jnp.where(is2, lane, float(LANES)), axis=1, keepdims=True)

    ind = jnp.where(lane == a1, 1.0, 0.0) + jnp.where(lane == a2, 1.0, 0.0)
    row = _fiota((TB, TB), 0)
    col = _fiota((TB, TB), 1)
    ltri = jnp.where(row >= col, 1.0, 0.0)
    incl = lax.dot(ltri, ind, precision=_HI)
    rank_ref[...] = incl - ind + carry[...]
    carry[...] = carry[...] + jnp.sum(ind, axis=0, keepdims=True)
    cnt_ref[...] = carry[...]

    a1_ref[...] = jnp.broadcast_to(a1, (TB, LANES))
    a2_ref[...] = jnp.broadcast_to(a2, (TB, LANES))
    w1o_ref[...] = jnp.broadcast_to(m1, (TB, LANES))
    w2o_ref[...] = jnp.broadcast_to(m2, (TB, LANES))


def _route_shared(x, gwt, sw1, sb1, sw3, sb3, sw2, sb2):
    f32 = jnp.float32
    out_shapes = (
        jax.ShapeDtypeStruct((T, D), f32),
        jax.ShapeDtypeStruct((T, LANES), f32),
        jax.ShapeDtypeStruct((T, LANES), f32),
        jax.ShapeDtypeStruct((T, LANES), f32),
        jax.ShapeDtypeStruct((T, LANES), f32),
        jax.ShapeDtypeStruct((T, LANES), f32),
        jax.ShapeDtypeStruct((1, LANES), f32),
    )
    blk = lambda i: (i, 0)
    whole = lambda i: (0, 0)
    return pl.pallas_call(
        _route_shared_body,
        grid=(NTB,),
        in_specs=[
            pl.BlockSpec((TB, D), blk),
            pl.BlockSpec((D, LANES), whole),
            pl.BlockSpec((D, F), whole),
            pl.BlockSpec((1, F), whole),
            pl.BlockSpec((D, F), whole),
            pl.BlockSpec((1, F), whole),
            pl.BlockSpec((F, D), whole),
            pl.BlockSpec((1, D), whole),
        ],
        out_specs=(
            pl.BlockSpec((TB, D), blk),
            pl.BlockSpec((TB, LANES), blk),
            pl.BlockSpec((TB, LANES), blk),
            pl.BlockSpec((TB, LANES), blk),
            pl.BlockSpec((TB, LANES), blk),
            pl.BlockSpec((TB, LANES), blk),
            pl.BlockSpec((1, LANES), whole),
        ),
        out_shape=out_shapes,
        scratch_shapes=[pltpu.VMEM((1, LANES), f32)],
        compiler_params=pltpu.CompilerParams(
            dimension_semantics=("arbitrary",)),
    )(x, gwt, sw1, sb1, sw3, sb3, sw2, sb2)


def _dispatch_meta_body(cnt_ref, a1_ref, a2_ref, rank_ref,
                        p0_ref, p1_ref, bem_ref, bvm_ref):
    c = cnt_ref[...]
    rc = jnp.floor((c + float(BT - 1)) / float(BT)) * float(BT)
    ri = _fiota((LANES, LANES), 0)
    ci = _fiota((LANES, LANES), 1)
    ut = jnp.where(ri <= ci, 1.0, 0.0)
    cum_incl = lax.dot(rc, ut, precision=_HI)
    cum_excl = cum_incl - rc

    rank = rank_ref[...]
    lane = _fiota((TB, LANES), 1)
    tot = cum_excl + rank
    pos0 = jnp.sum(jnp.where(lane == a1_ref[...], tot, 0.0),
                   axis=1, keepdims=True)
    pos1 = jnp.sum(jnp.where(lane == a2_ref[...], tot, 0.0),
                   axis=1, keepdims=True)
    p0_ref[...] = jnp.broadcast_to(pos0, (TB, LANES)).astype(jnp.int32)
    p1_ref[...] = jnp.broadcast_to(pos1, (TB, LANES)).astype(jnp.int32)

    start = ri * float(BT)
    lane8 = ci < E
    endb = jnp.broadcast_to(cum_incl, (LANES, LANES))
    be = jnp.sum(jnp.where(jnp.logical_and(lane8, start >= endb), 1.0, 0.0),
                 axis=1, keepdims=True)
    be = jnp.minimum(be, float(E - 1))
    bem_ref[...] = jnp.broadcast_to(be, (LANES, LANES)).astype(jnp.int32)
    exb = jnp.broadcast_to(cum_excl, (LANES, LANES))
    cb = jnp.broadcast_to(c, (LANES, LANES))
    has = jnp.logical_and(start >= exb, start < exb + cb)
    bv = jnp.sum(jnp.where(jnp.logical_and(lane8, has), 1.0, 0.0),
                 axis=1, keepdims=True)
    bvm_ref[...] = jnp.broadcast_to(bv, (LANES, LANES)).astype(jnp.int32)


def _dispatch_meta(cnt, a1, a2, rank):
    i32 = jnp.int32
    blk = lambda i: (i, 0)
    whole = lambda i: (0, 0)
    return pl.pallas_call(
        _dispatch_meta_body,
        grid=(NTB,),
        in_specs=[
            pl.BlockSpec((1, LANES), whole),
            pl.BlockSpec((TB, LANES), blk),
            pl.BlockSpec((TB, LANES), blk),
            pl.BlockSpec((TB, LANES), blk),
        ],
        out_specs=(
            pl.BlockSpec((TB, LANES), blk),
            pl.BlockSpec((TB, LANES), blk),
            pl.BlockSpec((LANES, LANES), whole),
            pl.BlockSpec((LANES, LANES), whole),
        ),
        out_shape=(
            jax.ShapeDtypeStruct((T, LANES), i32),
            jax.ShapeDtypeStruct((T, LANES), i32),
            jax.ShapeDtypeStruct((LANES, LANES), i32),
            jax.ShapeDtypeStruct((LANES, LANES), i32),
        ),
        compiler_params=pltpu.CompilerParams(
            dimension_semantics=("arbitrary",)),
    )(cnt, a1, a2, rank)


def _scatter_body(x_hbm, p0_hbm, p1_hbm, xs_hbm, xv, i0, i1, sem):
    wid = lax.axis_index("s") * NC + lax.axis_index("c")
    base = wid * TPW
    pltpu.sync_copy(x_hbm.at[pl.ds(base, TPW)], xv)
    pltpu.sync_copy(p0_hbm.at[pl.ds(base, TPW)], i0)
    pltpu.sync_copy(p1_hbm.at[pl.ds(base, TPW)], i1)
    pltpu.async_copy(xv, xs_hbm.at[i0], sem).wait()
    pltpu.async_copy(xv, xs_hbm.at[i1], sem).wait()


def _scatter_sorted(x, p0, p1):
    mesh = plsc.VectorSubcoreMesh(core_axis_name="c", subcore_axis_name="s")
    return pl.kernel(
        _scatter_body,
        out_type=jax.ShapeDtypeStruct((GR, D), jnp.float32),
        mesh=mesh,
        scratch_types=[
            pltpu.VMEM((TPW, D), jnp.float32),
            pltpu.VMEM((TPW,), jnp.int32),
            pltpu.VMEM((TPW,), jnp.int32),
            pltpu.SemaphoreType.DMA,
        ],
    )(x, p0, p1)


def _group_ffn_body(be_ref, bv_ref, xs_ref, w1_ref, b1_ref, w3_ref, b3_ref,
                    w2_ref, b2_ref, o_ref):
    i = pl.program_id(0)

    @pl.when(bv_ref[i] > 0)
    def _():
        x = xs_ref[...].astype(jnp.bfloat16)
        h1 = jnp.dot(x, w1_ref[0], preferred_element_type=jnp.float32)
        h1 = h1 + b1_ref[0]
        h3 = jnp.dot(x, w3_ref[0], preferred_element_type=jnp.float32)
        h3 = h3 + b3_ref[0]
        hh = (h1 * jax.nn.sigmoid(h1)) * h3
        o = jnp.dot(hh.astype(jnp.bfloat16), w2_ref[0],
                    preferred_element_type=jnp.float32)
        o_ref[...] = o + b2_ref[0]


def _group_ffn(bearr, bvarr, xs, ew1, eb1, ew3, eb3, ew2, eb2):
    grid_spec = pltpu.PrefetchScalarGridSpec(
        num_scalar_prefetch=2,
        grid=(NBR,),
        in_specs=[
            pl.BlockSpec((BT, D), lambda i, be, bv: (i, 0)),
            pl.BlockSpec((1, D, F), lambda i, be, bv: (be[i], 0, 0)),
            pl.BlockSpec((1, 1, F), lambda i, be, bv: (be[i], 0, 0)),
            pl.BlockSpec((1, D, F), lambda i, be, bv: (be[i], 0, 0)),
            pl.BlockSpec((1, 1, F), lambda i, be, bv: (be[i], 0, 0)),
            pl.BlockSpec((1, F, D), lambda i, be, bv: (be[i], 0, 0)),
            pl.BlockSpec((1, 1, D), lambda i, be, bv: (be[i], 0, 0)),
        ],
        out_specs=pl.BlockSpec((BT, D), lambda i, be, bv: (i, 0)),
    )
    return pl.pallas_call(
        _group_ffn_body,
        grid_spec=grid_spec,
        out_shape=jax.ShapeDtypeStruct((GR, D), jnp.float32),
        compiler_params=pltpu.CompilerParams(
            dimension_semantics=("arbitrary",)),
    )(bearr, bvarr, xs, ew1, eb1.reshape(E, 1, F), ew3,
      eb3.reshape(E, 1, F), ew2, eb2.reshape(E, 1, D))


def _combine_body(os_hbm, z_hbm, p0_hbm, p1_hbm, w0_hbm, w1_hbm, y_hbm,
                  g0, g1, gz, yv, i0c, i1c, w0m, w1m, sem):
    wid = lax.axis_index("s") * NC + lax.axis_index("c")
    base = wid * TPW
    for ch in range(TPW // DCH):
        tb = base + ch * DCH
        pltpu.sync_copy(p0_hbm.at[pl.ds(tb, DCH)], i0c)
        pltpu.sync_copy(p1_hbm.at[pl.ds(tb, DCH)], i1c)
        pltpu.sync_copy(w0_hbm.at[pl.ds(tb, DCH)], w0m)
        pltpu.sync_copy(w1_hbm.at[pl.ds(tb, DCH)], w1m)
        pltpu.async_copy(os_hbm.at[i0c], g0, sem).wait()
        pltpu.async_copy(os_hbm.at[i1c], g1, sem).wait()
        pltpu.sync_copy(z_hbm.at[pl.ds(tb, DCH)], gz)
        for t in range(DCH):
            w0b = w0m[t, :]
            w1b = w1m[t, :]

            def body(j, _):
                sl = pl.ds(j * 16, 16)
                yv[t, sl] = w0b * g0[t, sl] + w1b * g1[t, sl] + gz[t, sl]
                return 0

            lax.fori_loop(0, D // 16, body, 0)
        pltpu.sync_copy(yv, y_hbm.at[pl.ds(tb, DCH)])


def _combine(outs, z, p0, p1, w0, w1):
    f32 = jnp.float32
    mesh = plsc.VectorSubcoreMesh(core_axis_name="c", subcore_axis_name="s")
    return pl.kernel(
        _combine_body,
        out_type=jax.ShapeDtypeStruct((T, D), f32),
        mesh=mesh,
        scratch_types=[
            pltpu.VMEM((DCH, D), f32),
            pltpu.VMEM((DCH, D), f32),
            pltpu.VMEM((DCH, D), f32),
            pltpu.VMEM((DCH, D), f32),
            pltpu.VMEM((DCH,), jnp.int32),
            pltpu.VMEM((DCH,), jnp.int32),
            pltpu.VMEM((DCH, 16), f32),
            pltpu.VMEM((DCH, 16), f32),
            pltpu.SemaphoreType.DMA,
        ],
    )(outs, z, p0, p1, w0, w1)


def kernel(x, gate_w, ew1, eb1, ew2, eb2, ew3, eb3, sw1, sb1, sw2, sb2,
           sw3, sb3):
    f32 = jnp.float32
    gwt = jnp.zeros((D, LANES), f32).at[:, :E].set(gate_w.T)
    bf16 = jnp.bfloat16
    z, a1, a2, w1t, w2t, rank, cnt = _route_shared(
        x, gwt, sw1.astype(bf16), sb1.reshape(1, F), sw3.astype(bf16),
        sb3.reshape(1, F), sw2.astype(bf16), sb2.reshape(1, D))
    p0f, p1f, bem, bvm = _dispatch_meta(cnt, a1, a2, rank)
    p0 = p0f[:, 0]
    p1 = p1f[:, 0]
    bearr = bem[:NBR, 0]
    bvarr = bvm[:NBR, 0]
    xs = _scatter_sorted(x, p0, p1)
    outs = _group_ffn(bearr, bvarr, xs, ew1.astype(bf16), eb1,
                      ew3.astype(bf16), eb3, ew2.astype(bf16), eb2)
    y = _combine(outs, z, p0, p1, w1t[:, :16], w2t[:, :16])
    return y


# speedup vs baseline: 1.2876x; 1.2876x over previous
import functools

import jax
import jax.numpy as jnp
from jax import lax
from jax.experimental import pallas as pl
from jax.experimental.pallas import tpu as pltpu
from jax.experimental.pallas import tpu_sc as plsc

T, D, F, E, K = 2048, 1024, 2048, 8, 2
LANES = 128
TB = 256
NTB = T // TB
BT = 256
NBR = (T * K + E * BT) // BT
GR = NBR * BT
NC, NS = 2, 16
NW = NC * NS
TPW = T // NW
DCH = 16

_HI = lax.Precision.HIGHEST


def _fiota(shape, dim):
    return lax.broadcasted_iota(jnp.int32, shape, dim).astype(jnp.float32)


def _route_shared_body(x_ref, gwt_ref, sw1_ref, sb1_ref, sw3_ref, sb3_ref,
                       sw2_ref, sb2_ref,
                       z_ref, a1_ref, a2_ref, w1o_ref, w2o_ref, rank_ref,
                       cnt_ref, carry):
    pid = pl.program_id(0)

    @pl.when(pid == 0)
    def _():
        carry[...] = jnp.zeros_like(carry)

    x = x_ref[...]
    h1 = jnp.dot(x, sw1_ref[...], preferred_element_type=jnp.float32)
    h1 = h1 + sb1_ref[...]
    h3 = jnp.dot(x, sw3_ref[...], preferred_element_type=jnp.float32)
    h3 = h3 + sb3_ref[...]
    hh = (h1 * jax.nn.sigmoid(h1)) * h3
    z = jnp.dot(hh, sw2_ref[...], preferred_element_type=jnp.float32)
    z_ref[...] = z + sb2_ref[...]

    s = jnp.dot(x, gwt_ref[...], preferred_element_type=jnp.float32)
    lane = _fiota((TB, LANES), 1)
    valid = lane < E
    sm = jnp.where(valid, s, -jnp.inf)
    mx = jnp.max(sm, axis=1, keepdims=True)
    ex = jnp.exp(sm - mx)
    p = ex / jnp.sum(ex, axis=1, keepdims=True)
    m1 = jnp.max(p, axis=1, keepdims=True)
    is1 = jnp.logical_and(p == m1, valid)
    a1 = jnp.min(jnp.where(is1, lane, float(LANES)), axis=1, keepdims=True)
    not1 = lane != a1
    p2 = jnp.where(jnp.logical_and(not1, valid), p, -1.0)
    m2 = jnp.max(p2, axis=1, keepdims=True)
    is2 = jnp.logical_and(p2 == m2, jnp.logical_and(valid, not1))
    a2 = jnp.min(jnp.where(is2, lane, float(LANES)), axis=1, keepdims=True)

    ind = jnp.where(lane == a1, 1.0, 0.0) + jnp.where(lane == a2, 1.0, 0.0)
    row = _fiota((TB, TB), 0)
    col = _fiota((TB, TB), 1)
    ltri = jnp.where(row >= col, 1.0, 0.0)
    incl = lax.dot(ltri, ind, precision=_HI)
    rank_ref[...] = incl - ind + carry[...]
    carry[...] = carry[...] + jnp.sum(ind, axis=0, keepdims=True)
    cnt_ref[...] = carry[...]

    a1_ref[...] = jnp.broadcast_to(a1, (TB, LANES))
    a2_ref[...] = jnp.broadcast_to(a2, (TB, LANES))
    w1o_ref[...] = jnp.broadcast_to(m1, (TB, LANES))
    w2o_ref[...] = jnp.broadcast_to(m2, (TB, LANES))


def _route_shared(x, gwt, sw1, sb1, sw3, sb3, sw2, sb2):
    f32 = jnp.float32
    out_shapes = (
        jax.ShapeDtypeStruct((T, D), f32),
        jax.ShapeDtypeStruct((T, LANES), f32),
        jax.ShapeDtypeStruct((T, LANES), f32),
        jax.ShapeDtypeStruct((T, LANES), f32),
        jax.ShapeDtypeStruct((T, LANES), f32),
        jax.ShapeDtypeStruct((T, LANES), f32),
        jax.ShapeDtypeStruct((1, LANES), f32),
    )
    blk = lambda i: (i, 0)
    whole = lambda i: (0, 0)
    return pl.pallas_call(
        _route_shared_body,
        grid=(NTB,),
        in_specs=[
            pl.BlockSpec((TB, D), blk),
            pl.BlockSpec((D, LANES), whole),
            pl.BlockSpec((D, F), whole),
            pl.BlockSpec((1, F), whole),
            pl.BlockSpec((D, F), whole),
            pl.BlockSpec((1, F), whole),
            pl.BlockSpec((F, D), whole),
            pl.BlockSpec((1, D), whole),
        ],
        out_specs=(
            pl.BlockSpec((TB, D), blk),
            pl.BlockSpec((TB, LANES), blk),
            pl.BlockSpec((TB, LANES), blk),
            pl.BlockSpec((TB, LANES), blk),
            pl.BlockSpec((TB, LANES), blk),
            pl.BlockSpec((TB, LANES), blk),
            pl.BlockSpec((1, LANES), whole),
        ),
        out_shape=out_shapes,
        scratch_shapes=[pltpu.VMEM((1, LANES), f32)],
        compiler_params=pltpu.CompilerParams(
            dimension_semantics=("arbitrary",)),
    )(x, gwt, sw1, sb1, sw3, sb3, sw2, sb2)


def _dispatch_meta_body(cnt_ref, a1_ref, a2_ref, rank_ref,
                        p0_ref, p1_ref, bem_ref, bvm_ref):
    c = cnt_ref[...]
    rc = jnp.floor((c + float(BT - 1)) / float(BT)) * float(BT)
    ri = _fiota((LANES, LANES), 0)
    ci = _fiota((LANES, LANES), 1)
    ut = jnp.where(ri <= ci, 1.0, 0.0)
    cum_incl = lax.dot(rc, ut, precision=_HI)
    cum_excl = cum_incl - rc

    rank = rank_ref[...]
    lane = _fiota((TB, LANES), 1)
    tot = cum_excl + rank
    pos0 = jnp.sum(jnp.where(lane == a1_ref[...], tot, 0.0),
                   axis=1, keepdims=True)
    pos1 = jnp.sum(jnp.where(lane == a2_ref[...], tot, 0.0),
                   axis=1, keepdims=True)
    p0_ref[...] = jnp.broadcast_to(pos0, (TB, LANES)).astype(jnp.int32)
    p1_ref[...] = jnp.broadcast_to(pos1, (TB, LANES)).astype(jnp.int32)

    start = ri * float(BT)
    lane8 = ci < E
    endb = jnp.broadcast_to(cum_incl, (LANES, LANES))
    be = jnp.sum(jnp.where(jnp.logical_and(lane8, start >= endb), 1.0, 0.0),
                 axis=1, keepdims=True)
    be = jnp.minimum(be, float(E - 1))
    bem_ref[...] = jnp.broadcast_to(be, (LANES, LANES)).astype(jnp.int32)
    exb = jnp.broadcast_to(cum_excl, (LANES, LANES))
    cb = jnp.broadcast_to(c, (LANES, LANES))
    has = jnp.logical_and(start >= exb, start < exb + cb)
    bv = jnp.sum(jnp.where(jnp.logical_and(lane8, has), 1.0, 0.0),
                 axis=1, keepdims=True)
    bvm_ref[...] = jnp.broadcast_to(bv, (LANES, LANES)).astype(jnp.int32)


def _dispatch_meta(cnt, a1, a2, rank):
    i32 = jnp.int32
    blk = lambda i: (i, 0)
    whole = lambda i: (0, 0)
    return pl.pallas_call(
        _dispatch_meta_body,
        grid=(NTB,),
        in_specs=[
            pl.BlockSpec((1, LANES), whole),
            pl.BlockSpec((TB, LANES), blk),
            pl.BlockSpec((TB, LANES), blk),
            pl.BlockSpec((TB, LANES), blk),
        ],
        out_specs=(
            pl.BlockSpec((TB, LANES), blk),
            pl.BlockSpec((TB, LANES), blk),
            pl.BlockSpec((LANES, LANES), whole),
            pl.BlockSpec((LANES, LANES), whole),
        ),
        out_shape=(
            jax.ShapeDtypeStruct((T, LANES), i32),
            jax.ShapeDtypeStruct((T, LANES), i32),
            jax.ShapeDtypeStruct((LANES, LANES), i32),
            jax.ShapeDtypeStruct((LANES, LANES), i32),
        ),
        compiler_params=pltpu.CompilerParams(
            dimension_semantics=("arbitrary",)),
    )(cnt, a1, a2, rank)


def _scatter_body(x_hbm, p0_hbm, p1_hbm, w0_hbm, w1_hbm, xs_hbm, ws_hbm,
                  xv, i0, i1, w0v, w1v, sem):
    wid = lax.axis_index("s") * NC + lax.axis_index("c")
    base = wid * TPW
    pltpu.sync_copy(x_hbm.at[pl.ds(base, TPW)], xv)
    pltpu.sync_copy(p0_hbm.at[pl.ds(base, TPW)], i0)
    pltpu.sync_copy(p1_hbm.at[pl.ds(base, TPW)], i1)
    pltpu.sync_copy(w0_hbm.at[pl.ds(base, TPW)], w0v)
    pltpu.sync_copy(w1_hbm.at[pl.ds(base, TPW)], w1v)
    a = pltpu.async_copy(xv, xs_hbm.at[i0], sem)
    b = pltpu.async_copy(xv, xs_hbm.at[i1], sem)
    c = pltpu.async_copy(w0v, ws_hbm.at[i0], sem)
    d = pltpu.async_copy(w1v, ws_hbm.at[i1], sem)
    a.wait()
    b.wait()
    c.wait()
    d.wait()


def _scatter_sorted(x, p0, p1, w0r, w1r):
    mesh = plsc.VectorSubcoreMesh(core_axis_name="c", subcore_axis_name="s")
    return pl.kernel(
        _scatter_body,
        out_type=(jax.ShapeDtypeStruct((GR, D), jnp.float32),
                  jax.ShapeDtypeStruct((GR, LANES), jnp.float32)),
        mesh=mesh,
        scratch_types=[
            pltpu.VMEM((TPW, D), jnp.float32),
            pltpu.VMEM((TPW,), jnp.int32),
            pltpu.VMEM((TPW,), jnp.int32),
            pltpu.VMEM((TPW, LANES), jnp.float32),
            pltpu.VMEM((TPW, LANES), jnp.float32),
            pltpu.SemaphoreType.DMA,
        ],
    )(x, p0, p1, w0r, w1r)


def _group_ffn_body(be_ref, bv_ref, xs_ref, ws_ref, w1_ref, b1_ref, w3_ref,
                    b3_ref, w2_ref, b2_ref, o_ref):
    i = pl.program_id(0)

    @pl.when(bv_ref[i] > 0)
    def _():
        x = xs_ref[...]
        h1 = jnp.dot(x, w1_ref[0], preferred_element_type=jnp.float32)
        h1 = h1 + b1_ref[0]
        h3 = jnp.dot(x, w3_ref[0], preferred_element_type=jnp.float32)
        h3 = h3 + b3_ref[0]
        hh = (h1 * jax.nn.sigmoid(h1)) * h3
        o = jnp.dot(hh, w2_ref[0], preferred_element_type=jnp.float32)
        o_ref[...] = (o + b2_ref[0]) * ws_ref[:, :1]


def _group_ffn(bearr, bvarr, xs, ws, ew1, eb1, ew3, eb3, ew2, eb2):
    grid_spec = pltpu.PrefetchScalarGridSpec(
        num_scalar_prefetch=2,
        grid=(NBR,),
        in_specs=[
            pl.BlockSpec((BT, D), lambda i, be, bv: (i, 0)),
            pl.BlockSpec((BT, LANES), lambda i, be, bv: (i, 0)),
            pl.BlockSpec((1, D, F), lambda i, be, bv: (be[i], 0, 0)),
            pl.BlockSpec((1, 1, F), lambda i, be, bv: (be[i], 0, 0)),
            pl.BlockSpec((1, D, F), lambda i, be, bv: (be[i], 0, 0)),
            pl.BlockSpec((1, 1, F), lambda i, be, bv: (be[i], 0, 0)),
            pl.BlockSpec((1, F, D), lambda i, be, bv: (be[i], 0, 0)),
            pl.BlockSpec((1, 1, D), lambda i, be, bv: (be[i], 0, 0)),
        ],
        out_specs=pl.BlockSpec((BT, D), lambda i, be, bv: (i, 0)),
    )
    return pl.pallas_call(
        _group_ffn_body,
        grid_spec=grid_spec,
        out_shape=jax.ShapeDtypeStruct((GR, D), jnp.float32),
        compiler_params=pltpu.CompilerParams(
            dimension_semantics=("arbitrary",)),
    )(bearr, bvarr, xs, ws, ew1, eb1.reshape(E, 1, F), ew3,
      eb3.reshape(E, 1, F), ew2, eb2.reshape(E, 1, D))


def _combine_body(os_hbm, z_hbm, p0_hbm, p1_hbm, y_hbm,
                  g0, g1, gz, yv, i0c, i1c, sem):
    wid = lax.axis_index("s") * NC + lax.axis_index("c")
    base = wid * TPW
    for ch in range(TPW // DCH):
        tb = base + ch * DCH
        pltpu.sync_copy(p0_hbm.at[pl.ds(tb, DCH)], i0c)
        pltpu.sync_copy(p1_hbm.at[pl.ds(tb, DCH)], i1c)
        a = pltpu.async_copy(os_hbm.at[i0c], g0, sem)
        b = pltpu.async_copy(os_hbm.at[i1c], g1, sem)
        pltpu.sync_copy(z_hbm.at[pl.ds(tb, DCH)], gz)
        a.wait()
        b.wait()
        for t in range(DCH):

            def body(j, _):
                for u in range(4):
                    sl = pl.ds(j * 64 + u * 16, 16)
                    yv[t, sl] = g0[t, sl] + g1[t, sl] + gz[t, sl]
                return 0

            lax.fori_loop(0, D // 64, body, 0)
        pltpu.sync_copy(yv, y_hbm.at[pl.ds(tb, DCH)])


def _combine(outs, z, p0, p1):
    f32 = jnp.float32
    mesh = plsc.VectorSubcoreMesh(core_axis_name="c", subcore_axis_name="s")
    return pl.kernel(
        _combine_body,
        out_type=jax.ShapeDtypeStruct((T, D), f32),
        mesh=mesh,
        scratch_types=[
            pltpu.VMEM((DCH, D), f32),
            pltpu.VMEM((DCH, D), f32),
            pltpu.VMEM((DCH, D), f32),
            pltpu.VMEM((DCH, D), f32),
            pltpu.VMEM((DCH,), jnp.int32),
            pltpu.VMEM((DCH,), jnp.int32),
            pltpu.SemaphoreType.DMA,
        ],
    )(outs, z, p0, p1)


def kernel(x, gate_w, ew1, eb1, ew2, eb2, ew3, eb3, sw1, sb1, sw2, sb2,
           sw3, sb3):
    f32 = jnp.float32
    gwt = jnp.zeros((D, LANES), f32).at[:, :E].set(gate_w.T)
    z, a1, a2, w1t, w2t, rank, cnt = _route_shared(
        x, gwt, sw1, sb1.reshape(1, F), sw3, sb3.reshape(1, F),
        sw2, sb2.reshape(1, D))
    p0f, p1f, bem, bvm = _dispatch_meta(cnt, a1, a2, rank)
    p0 = p0f[:, 0]
    p1 = p1f[:, 0]
    bearr = bem[:NBR, 0]
    bvarr = bvm[:NBR, 0]
    xs, ws = _scatter_sorted(x, p0, p1, w1t, w2t)
    outs = _group_ffn(bearr, bvarr, xs, ws, ew1, eb1, ew3, eb3, ew2, eb2)
    y = _combine(outs, z, p0, p1)
    return y


# speedup vs baseline: 1.3013x; 1.0106x over previous
import functools

import jax
import jax.numpy as jnp
from jax import lax
from jax.experimental import pallas as pl
from jax.experimental.pallas import tpu as pltpu
from jax.experimental.pallas import tpu_sc as plsc

T, D, F, E, K = 2048, 1024, 2048, 8, 2
LANES = 128
TB = 256
NTB = T // TB
BT = 384
NBR = -(-(T * K + E * (BT - 1)) // BT)
GR = NBR * BT
NC, NS = 2, 16
NW = NC * NS
TPW = T // NW
DCH = 16

_HI = lax.Precision.HIGHEST


def _fiota(shape, dim):
    return lax.broadcasted_iota(jnp.int32, shape, dim).astype(jnp.float32)


def _route_shared_body(x_ref, gwt_ref, sw1_ref, sb1_ref, sw3_ref, sb3_ref,
                       sw2_ref, sb2_ref,
                       z_ref, a1_ref, a2_ref, w1o_ref, w2o_ref, rank_ref,
                       cnt_ref, carry):
    pid = pl.program_id(0)

    @pl.when(pid == 0)
    def _():
        carry[...] = jnp.zeros_like(carry)

    x = x_ref[...]
    h1 = jnp.dot(x, sw1_ref[...], preferred_element_type=jnp.float32)
    h1 = h1 + sb1_ref[...]
    h3 = jnp.dot(x, sw3_ref[...], preferred_element_type=jnp.float32)
    h3 = h3 + sb3_ref[...]
    hh = (h1 * jax.nn.sigmoid(h1)) * h3
    z = jnp.dot(hh, sw2_ref[...], preferred_element_type=jnp.float32)
    z_ref[...] = z + sb2_ref[...]

    s = jnp.dot(x, gwt_ref[...], preferred_element_type=jnp.float32)
    lane = _fiota((TB, LANES), 1)
    valid = lane < E
    sm = jnp.where(valid, s, -jnp.inf)
    mx = jnp.max(sm, axis=1, keepdims=True)
    ex = jnp.exp(sm - mx)
    p = ex / jnp.sum(ex, axis=1, keepdims=True)
    m1 = jnp.max(p, axis=1, keepdims=True)
    is1 = jnp.logical_and(p == m1, valid)
    a1 = jnp.min(jnp.where(is1, lane, float(LANES)), axis=1, keepdims=True)
    not1 = lane != a1
    p2 = jnp.where(jnp.logical_and(not1, valid), p, -1.0)
    m2 = jnp.max(p2, axis=1, keepdims=True)
    is2 = jnp.logical_and(p2 == m2, jnp.logical_and(valid, not1))
    a2 = jnp.min(jnp.where(is2, lane, float(LANES)), axis=1, keepdims=True)

    ind = jnp.where(lane == a1, 1.0, 0.0) + jnp.where(lane == a2, 1.0, 0.0)
    row = _fiota((TB, TB), 0)
    col = _fiota((TB, TB), 1)
    ltri = jnp.where(row >= col, 1.0, 0.0)
    incl = lax.dot(ltri, ind, precision=_HI)
    rank_ref[...] = incl - ind + carry[...]
    carry[...] = carry[...] + jnp.sum(ind, axis=0, keepdims=True)
    cnt_ref[...] = carry[...]

    a1_ref[...] = jnp.broadcast_to(a1, (TB, LANES))
    a2_ref[...] = jnp.broadcast_to(a2, (TB, LANES))
    w1o_ref[...] = jnp.broadcast_to(m1, (TB, LANES))
    w2o_ref[...] = jnp.broadcast_to(m2, (TB, LANES))


def _route_shared(x, gwt, sw1, sb1, sw3, sb3, sw2, sb2):
    f32 = jnp.float32
    out_shapes = (
        jax.ShapeDtypeStruct((T, D), f32),
        jax.ShapeDtypeStruct((T, LANES), f32),
        jax.ShapeDtypeStruct((T, LANES), f32),
        jax.ShapeDtypeStruct((T, LANES), f32),
        jax.ShapeDtypeStruct((T, LANES), f32),
        jax.ShapeDtypeStruct((T, LANES), f32),
        jax.ShapeDtypeStruct((1, LANES), f32),
    )
    blk = lambda i: (i, 0)
    whole = lambda i: (0, 0)
    return pl.pallas_call(
        _route_shared_body,
        grid=(NTB,),
        in_specs=[
            pl.BlockSpec((TB, D), blk),
            pl.BlockSpec((D, LANES), whole),
            pl.BlockSpec((D, F), whole),
            pl.BlockSpec((1, F), whole),
            pl.BlockSpec((D, F), whole),
            pl.BlockSpec((1, F), whole),
            pl.BlockSpec((F, D), whole),
            pl.BlockSpec((1, D), whole),
        ],
        out_specs=(
            pl.BlockSpec((TB, D), blk),
            pl.BlockSpec((TB, LANES), blk),
            pl.BlockSpec((TB, LANES), blk),
            pl.BlockSpec((TB, LANES), blk),
            pl.BlockSpec((TB, LANES), blk),
            pl.BlockSpec((TB, LANES), blk),
            pl.BlockSpec((1, LANES), whole),
        ),
        out_shape=out_shapes,
        scratch_shapes=[pltpu.VMEM((1, LANES), f32)],
        compiler_params=pltpu.CompilerParams(
            dimension_semantics=("arbitrary",)),
    )(x, gwt, sw1, sb1, sw3, sb3, sw2, sb2)


def _dispatch_meta_body(cnt_ref, a1_ref, a2_ref, rank_ref,
                        p0_ref, p1_ref, bem_ref, bvm_ref):
    c = cnt_ref[...]
    rc = jnp.floor((c + float(BT - 1)) / float(BT)) * float(BT)
    ri = _fiota((LANES, LANES), 0)
    ci = _fiota((LANES, LANES), 1)
    ut = jnp.where(ri <= ci, 1.0, 0.0)
    cum_incl = lax.dot(rc, ut, precision=_HI)
    cum_excl = cum_incl - rc

    rank = rank_ref[...]
    lane = _fiota((TB, LANES), 1)
    tot = cum_excl + rank
    pos0 = jnp.sum(jnp.where(lane == a1_ref[...], tot, 0.0),
                   axis=1, keepdims=True)
    pos1 = jnp.sum(jnp.where(lane == a2_ref[...], tot, 0.0),
                   axis=1, keepdims=True)
    p0_ref[...] = jnp.broadcast_to(pos0, (TB, LANES)).astype(jnp.int32)
    p1_ref[...] = jnp.broadcast_to(pos1, (TB, LANES)).astype(jnp.int32)

    @pl.when(pl.program_id(0) == 0)
    def _():
        start = ri * float(BT)
        lane8 = ci < E
        endb = jnp.broadcast_to(cum_incl, (LANES, LANES))
        be = jnp.sum(jnp.where(jnp.logical_and(lane8, start >= endb),
                               1.0, 0.0), axis=1, keepdims=True)
        be = jnp.minimum(be, float(E - 1))
        bem_ref[...] = jnp.broadcast_to(be, (LANES, LANES)).astype(jnp.int32)
        exb = jnp.broadcast_to(cum_excl, (LANES, LANES))
        cb = jnp.broadcast_to(c, (LANES, LANES))
        has = jnp.logical_and(start >= exb, start < exb + cb)
        bv = jnp.sum(jnp.where(jnp.logical_and(lane8, has), 1.0, 0.0),
                     axis=1, keepdims=True)
        bvm_ref[...] = jnp.broadcast_to(bv, (LANES, LANES)).astype(jnp.int32)


def _dispatch_meta(cnt, a1, a2, rank):
    i32 = jnp.int32
    blk = lambda i: (i, 0)
    whole = lambda i: (0, 0)
    return pl.pallas_call(
        _dispatch_meta_body,
        grid=(NTB,),
        in_specs=[
            pl.BlockSpec((1, LANES), whole),
            pl.BlockSpec((TB, LANES), blk),
            pl.BlockSpec((TB, LANES), blk),
            pl.BlockSpec((TB, LANES), blk),
        ],
        out_specs=(
            pl.BlockSpec((TB, LANES), blk),
            pl.BlockSpec((TB, LANES), blk),
            pl.BlockSpec((LANES, LANES), whole),
            pl.BlockSpec((LANES, LANES), whole),
        ),
        out_shape=(
            jax.ShapeDtypeStruct((T, LANES), i32),
            jax.ShapeDtypeStruct((T, LANES), i32),
            jax.ShapeDtypeStruct((LANES, LANES), i32),
            jax.ShapeDtypeStruct((LANES, LANES), i32),
        ),
        compiler_params=pltpu.CompilerParams(
            dimension_semantics=("arbitrary",)),
    )(cnt, a1, a2, rank)


def _scatter_body(x_hbm, p0_hbm, p1_hbm, w0_hbm, w1_hbm, xs_hbm, ws_hbm,
                  xv, i0, i1, w0v, w1v, sem):
    wid = lax.axis_index("s") * NC + lax.axis_index("c")
    base = wid * TPW
    pltpu.sync_copy(x_hbm.at[pl.ds(base, TPW)], xv)
    pltpu.sync_copy(p0_hbm.at[pl.ds(base, TPW)], i0)
    pltpu.sync_copy(p1_hbm.at[pl.ds(base, TPW)], i1)
    pltpu.sync_copy(w0_hbm.at[pl.ds(base, TPW)], w0v)
    pltpu.sync_copy(w1_hbm.at[pl.ds(base, TPW)], w1v)
    a = pltpu.async_copy(xv, xs_hbm.at[i0], sem)
    b = pltpu.async_copy(xv, xs_hbm.at[i1], sem)
    c = pltpu.async_copy(w0v, ws_hbm.at[i0], sem)
    d = pltpu.async_copy(w1v, ws_hbm.at[i1], sem)
    a.wait()
    b.wait()
    c.wait()
    d.wait()


def _scatter_sorted(x, p0, p1, w0r, w1r):
    mesh = plsc.VectorSubcoreMesh(core_axis_name="c", subcore_axis_name="s")
    return pl.kernel(
        _scatter_body,
        out_type=(jax.ShapeDtypeStruct((GR, D), jnp.float32),
                  jax.ShapeDtypeStruct((GR, LANES), jnp.float32)),
        mesh=mesh,
        scratch_types=[
            pltpu.VMEM((TPW, D), jnp.float32),
            pltpu.VMEM((TPW,), jnp.int32),
            pltpu.VMEM((TPW,), jnp.int32),
            pltpu.VMEM((TPW, LANES), jnp.float32),
            pltpu.VMEM((TPW, LANES), jnp.float32),
            pltpu.SemaphoreType.DMA,
        ],
    )(x, p0, p1, w0r, w1r)


def _group_ffn_body(be_ref, bv_ref, xs_ref, ws_ref, w1_ref, b1_ref, w3_ref,
                    b3_ref, w2_ref, b2_ref, o_ref):
    i = pl.program_id(0)

    @pl.when(bv_ref[i] > 0)
    def _():
        x = xs_ref[...]
        h1 = jnp.dot(x, w1_ref[0], preferred_element_type=jnp.float32)
        h1 = h1 + b1_ref[0]
        h3 = jnp.dot(x, w3_ref[0], preferred_element_type=jnp.float32)
        h3 = h3 + b3_ref[0]
        hh = (h1 * jax.nn.sigmoid(h1)) * h3
        o = jnp.dot(hh, w2_ref[0], preferred_element_type=jnp.float32)
        o_ref[...] = (o + b2_ref[0]) * ws_ref[:, :1]


def _group_ffn(bearr, bvarr, xs, ws, ew1, eb1, ew3, eb3, ew2, eb2):
    grid_spec = pltpu.PrefetchScalarGridSpec(
        num_scalar_prefetch=2,
        grid=(NBR,),
        in_specs=[
            pl.BlockSpec((BT, D), lambda i, be, bv: (i, 0)),
            pl.BlockSpec((BT, LANES), lambda i, be, bv: (i, 0)),
            pl.BlockSpec((1, D, F), lambda i, be, bv: (be[i], 0, 0)),
            pl.BlockSpec((1, 1, F), lambda i, be, bv: (be[i], 0, 0)),
            pl.BlockSpec((1, D, F), lambda i, be, bv: (be[i], 0, 0)),
            pl.BlockSpec((1, 1, F), lambda i, be, bv: (be[i], 0, 0)),
            pl.BlockSpec((1, F, D), lambda i, be, bv: (be[i], 0, 0)),
            pl.BlockSpec((1, 1, D), lambda i, be, bv: (be[i], 0, 0)),
        ],
        out_specs=pl.BlockSpec((BT, D), lambda i, be, bv: (i, 0)),
    )
    return pl.pallas_call(
        _group_ffn_body,
        grid_spec=grid_spec,
        out_shape=jax.ShapeDtypeStruct((GR, D), jnp.float32),
        compiler_params=pltpu.CompilerParams(
            dimension_semantics=("arbitrary",)),
    )(bearr, bvarr, xs, ws, ew1, eb1.reshape(E, 1, F), ew3,
      eb3.reshape(E, 1, F), ew2, eb2.reshape(E, 1, D))


def _combine_body(os_hbm, z_hbm, p0_hbm, p1_hbm, y_hbm,
                  g0a, g1a, gza, g0b, g1b, gzb, yv, i0a, i1a, i0b, i1b,
                  sema, semb):
    wid = lax.axis_index("s") * NC + lax.axis_index("c")
    base = wid * TPW
    bufs = ((g0a, g1a, gza, i0a, i1a, sema),
            (g0b, g1b, gzb, i0b, i1b, semb))

    def issue(ch, g0, g1, gz, i0, i1, sem):
        tb = base + ch * DCH
        pltpu.sync_copy(p0_hbm.at[pl.ds(tb, DCH)], i0)
        pltpu.sync_copy(p1_hbm.at[pl.ds(tb, DCH)], i1)
        return (pltpu.async_copy(os_hbm.at[i0], g0, sem),
                pltpu.async_copy(os_hbm.at[i1], g1, sem),
                pltpu.async_copy(z_hbm.at[pl.ds(tb, DCH)], gz, sem))

    nch = TPW // DCH
    pend = issue(0, *bufs[0])
    for ch in range(nch):
        g0, g1, gz = bufs[ch % 2][:3]
        for cp in pend:
            cp.wait()
        if ch + 1 < nch:
            pend = issue(ch + 1, *bufs[(ch + 1) % 2])
        for t in range(DCH):

            def body(j, _):
                for u in range(4):
                    sl = pl.ds(j * 64 + u * 16, 16)
                    yv[t, sl] = g0[t, sl] + g1[t, sl] + gz[t, sl]
                return 0

            lax.fori_loop(0, D // 64, body, 0)
        pltpu.sync_copy(yv, y_hbm.at[pl.ds(base + ch * DCH, DCH)])


def _combine(outs, z, p0, p1):
    f32 = jnp.float32
    mesh = plsc.VectorSubcoreMesh(core_axis_name="c", subcore_axis_name="s")
    return pl.kernel(
        _combine_body,
        out_type=jax.ShapeDtypeStruct((T, D), f32),
        mesh=mesh,
        scratch_types=[
            pltpu.VMEM((DCH, D), f32),
            pltpu.VMEM((DCH, D), f32),
            pltpu.VMEM((DCH, D), f32),
            pltpu.VMEM((DCH, D), f32),
            pltpu.VMEM((DCH, D), f32),
            pltpu.VMEM((DCH, D), f32),
            pltpu.VMEM((DCH, D), f32),
            pltpu.VMEM((DCH,), jnp.int32),
            pltpu.VMEM((DCH,), jnp.int32),
            pltpu.VMEM((DCH,), jnp.int32),
            pltpu.VMEM((DCH,), jnp.int32),
            pltpu.SemaphoreType.DMA,
            pltpu.SemaphoreType.DMA,
        ],
    )(outs, z, p0, p1)


def kernel(x, gate_w, ew1, eb1, ew2, eb2, ew3, eb3, sw1, sb1, sw2, sb2,
           sw3, sb3):
    f32 = jnp.float32
    gwt = jnp.zeros((D, LANES), f32).at[:, :E].set(gate_w.T)
    z, a1, a2, w1t, w2t, rank, cnt = _route_shared(
        x, gwt, sw1, sb1.reshape(1, F), sw3, sb3.reshape(1, F),
        sw2, sb2.reshape(1, D))
    p0f, p1f, bem, bvm = _dispatch_meta(cnt, a1, a2, rank)
    p0 = p0f[:, 0]
    p1 = p1f[:, 0]
    bearr = bem[:NBR, 0]
    bvarr = bvm[:NBR, 0]
    xs, ws = _scatter_sorted(x, p0, p1, w1t, w2t)
    outs = _group_ffn(bearr, bvarr, xs, ws, ew1, eb1, ew3, eb3, ew2, eb2)
    y = _combine(outs, z, p0, p1)
    return y


# speedup vs baseline: 1.5001x; 1.1528x over previous
import functools

import jax
import jax.numpy as jnp
from jax import lax
from jax.experimental import pallas as pl
from jax.experimental.pallas import tpu as pltpu
from jax.experimental.pallas import tpu_sc as plsc

T, D, F, E, K = 2048, 1024, 2048, 8, 2
LANES = 128
TB = 256
NTB = T // TB
BT = 256
NBR = -(-(T * K + E * (BT - 1)) // BT)
GR = NBR * BT
NC, NS = 2, 16
NW = NC * NS
TPW = T // NW
DCH = 16

_HI = lax.Precision.HIGHEST


def _fiota(shape, dim):
    return lax.broadcasted_iota(jnp.int32, shape, dim).astype(jnp.float32)


def _route_shared_body(x_ref, gwt_ref, sw1_ref, sb1_ref, sw3_ref, sb3_ref,
                       sw2_ref, sb2_ref,
                       z_ref, a1_ref, a2_ref, w1o_ref, w2o_ref, rank_ref,
                       cnt_ref, carry):
    pid = pl.program_id(0)

    @pl.when(pid == 0)
    def _():
        carry[...] = jnp.zeros_like(carry)

    x = x_ref[...]
    h1 = jnp.dot(x, sw1_ref[...], preferred_element_type=jnp.float32)
    h1 = h1 + sb1_ref[...]
    h3 = jnp.dot(x, sw3_ref[...], preferred_element_type=jnp.float32)
    h3 = h3 + sb3_ref[...]
    hh = (h1 * jax.nn.sigmoid(h1)) * h3
    z = jnp.dot(hh, sw2_ref[...], preferred_element_type=jnp.float32)
    z_ref[...] = z + sb2_ref[...]

    s = jnp.dot(x, gwt_ref[...], preferred_element_type=jnp.float32)
    lane = _fiota((TB, LANES), 1)
    valid = lane < E
    sm = jnp.where(valid, s, -jnp.inf)
    mx = jnp.max(sm, axis=1, keepdims=True)
    ex = jnp.exp(sm - mx)
    p = ex / jnp.sum(ex, axis=1, keepdims=True)
    m1 = jnp.max(p, axis=1, keepdims=True)
    is1 = jnp.logical_and(p == m1, valid)
    a1 = jnp.min(jnp.where(is1, lane, float(LANES)), axis=1, keepdims=True)
    not1 = lane != a1
    p2 = jnp.where(jnp.logical_and(not1, valid), p, -1.0)
    m2 = jnp.max(p2, axis=1, keepdims=True)
    is2 = jnp.logical_and(p2 == m2, jnp.logical_and(valid, not1))
    a2 = jnp.min(jnp.where(is2, lane, float(LANES)), axis=1, keepdims=True)

    ind = jnp.where(lane == a1, 1.0, 0.0) + jnp.where(lane == a2, 1.0, 0.0)
    row = _fiota((TB, TB), 0)
    col = _fiota((TB, TB), 1)
    ltri = jnp.where(row >= col, 1.0, 0.0)
    incl = lax.dot(ltri, ind, precision=_HI)
    rank_ref[...] = incl - ind + carry[...]
    carry[...] = carry[...] + jnp.sum(ind, axis=0, keepdims=True)
    cnt_ref[...] = carry[...]

    a1_ref[...] = jnp.broadcast_to(a1, (TB, LANES))
    a2_ref[...] = jnp.broadcast_to(a2, (TB, LANES))
    w1o_ref[...] = jnp.broadcast_to(m1, (TB, LANES))
    w2o_ref[...] = jnp.broadcast_to(m2, (TB, LANES))


def _route_shared(x, gwt, sw1, sb1, sw3, sb3, sw2, sb2):
    f32 = jnp.float32
    out_shapes = (
        jax.ShapeDtypeStruct((T, D), f32),
        jax.ShapeDtypeStruct((T, LANES), f32),
        jax.ShapeDtypeStruct((T, LANES), f32),
        jax.ShapeDtypeStruct((T, LANES), f32),
        jax.ShapeDtypeStruct((T, LANES), f32),
        jax.ShapeDtypeStruct((T, LANES), f32),
        jax.ShapeDtypeStruct((1, LANES), f32),
    )
    blk = lambda i: (i, 0)
    whole = lambda i: (0, 0)
    return pl.pallas_call(
        _route_shared_body,
        grid=(NTB,),
        in_specs=[
            pl.BlockSpec((TB, D), blk),
            pl.BlockSpec((D, LANES), whole),
            pl.BlockSpec((D, F), whole),
            pl.BlockSpec((1, F), whole),
            pl.BlockSpec((D, F), whole),
            pl.BlockSpec((1, F), whole),
            pl.BlockSpec((F, D), whole),
            pl.BlockSpec((1, D), whole),
        ],
        out_specs=(
            pl.BlockSpec((TB, D), blk),
            pl.BlockSpec((TB, LANES), blk),
            pl.BlockSpec((TB, LANES), blk),
            pl.BlockSpec((TB, LANES), blk),
            pl.BlockSpec((TB, LANES), blk),
            pl.BlockSpec((TB, LANES), blk),
            pl.BlockSpec((1, LANES), whole),
        ),
        out_shape=out_shapes,
        scratch_shapes=[pltpu.VMEM((1, LANES), f32)],
        compiler_params=pltpu.CompilerParams(
            dimension_semantics=("arbitrary",)),
    )(x, gwt, sw1, sb1, sw3, sb3, sw2, sb2)


def _dispatch_meta_body(cnt_ref, a1_ref, a2_ref, rank_ref,
                        p0_ref, p1_ref, bem_ref, bvm_ref):
    c = cnt_ref[...]
    rc = jnp.floor((c + float(BT - 1)) / float(BT)) * float(BT)
    ri = _fiota((LANES, LANES), 0)
    ci = _fiota((LANES, LANES), 1)
    ut = jnp.where(ri <= ci, 1.0, 0.0)
    cum_incl = lax.dot(rc, ut, precision=_HI)
    cum_excl = cum_incl - rc

    rank = rank_ref[...]
    lane = _fiota((TB, LANES), 1)
    tot = cum_excl + rank
    pos0 = jnp.sum(jnp.where(lane == a1_ref[...], tot, 0.0),
                   axis=1, keepdims=True)
    pos1 = jnp.sum(jnp.where(lane == a2_ref[...], tot, 0.0),
                   axis=1, keepdims=True)
    p0_ref[...] = jnp.broadcast_to(pos0, (TB, LANES)).astype(jnp.int32)
    p1_ref[...] = jnp.broadcast_to(pos1, (TB, LANES)).astype(jnp.int32)

    @pl.when(pl.program_id(0) == 0)
    def _():
        start = ri * float(BT)
        lane8 = ci < E
        endb = jnp.broadcast_to(cum_incl, (LANES, LANES))
        be = jnp.sum(jnp.where(jnp.logical_and(lane8, start >= endb),
                               1.0, 0.0), axis=1, keepdims=True)
        be = jnp.minimum(be, float(E - 1))
        bem_ref[...] = jnp.broadcast_to(be, (LANES, LANES)).astype(jnp.int32)
        exb = jnp.broadcast_to(cum_excl, (LANES, LANES))
        cb = jnp.broadcast_to(c, (LANES, LANES))
        has = jnp.logical_and(start >= exb, start < exb + cb)
        bv = jnp.sum(jnp.where(jnp.logical_and(lane8, has), 1.0, 0.0),
                     axis=1, keepdims=True)
        bvm_ref[...] = jnp.broadcast_to(bv, (LANES, LANES)).astype(jnp.int32)


def _dispatch_meta(cnt, a1, a2, rank):
    i32 = jnp.int32
    blk = lambda i: (i, 0)
    whole = lambda i: (0, 0)
    return pl.pallas_call(
        _dispatch_meta_body,
        grid=(NTB,),
        in_specs=[
            pl.BlockSpec((1, LANES), whole),
            pl.BlockSpec((TB, LANES), blk),
            pl.BlockSpec((TB, LANES), blk),
            pl.BlockSpec((TB, LANES), blk),
        ],
        out_specs=(
            pl.BlockSpec((TB, LANES), blk),
            pl.BlockSpec((TB, LANES), blk),
            pl.BlockSpec((LANES, LANES), whole),
            pl.BlockSpec((LANES, LANES), whole),
        ),
        out_shape=(
            jax.ShapeDtypeStruct((T, LANES), i32),
            jax.ShapeDtypeStruct((T, LANES), i32),
            jax.ShapeDtypeStruct((LANES, LANES), i32),
            jax.ShapeDtypeStruct((LANES, LANES), i32),
        ),
        compiler_params=pltpu.CompilerParams(
            dimension_semantics=("arbitrary",)),
    )(cnt, a1, a2, rank)


def _scatter_body(x_hbm, p0_hbm, p1_hbm, w0_hbm, w1_hbm, xs_hbm, ws_hbm,
                  xv, i0, i1, w0v, w1v, sem):
    wid = lax.axis_index("s") * NC + lax.axis_index("c")
    base = wid * TPW
    pltpu.sync_copy(x_hbm.at[pl.ds(base, TPW)], xv)
    pltpu.sync_copy(p0_hbm.at[pl.ds(base, TPW)], i0)
    pltpu.sync_copy(p1_hbm.at[pl.ds(base, TPW)], i1)
    pltpu.sync_copy(w0_hbm.at[pl.ds(base, TPW)], w0v)
    pltpu.sync_copy(w1_hbm.at[pl.ds(base, TPW)], w1v)
    a = pltpu.async_copy(xv, xs_hbm.at[i0], sem)
    b = pltpu.async_copy(xv, xs_hbm.at[i1], sem)
    c = pltpu.async_copy(w0v, ws_hbm.at[i0], sem)
    d = pltpu.async_copy(w1v, ws_hbm.at[i1], sem)
    a.wait()
    b.wait()
    c.wait()
    d.wait()


def _scatter_sorted(x, p0, p1, w0r, w1r):
    mesh = plsc.VectorSubcoreMesh(core_axis_name="c", subcore_axis_name="s")
    return pl.kernel(
        _scatter_body,
        out_type=(jax.ShapeDtypeStruct((GR, D), jnp.float32),
                  jax.ShapeDtypeStruct((GR, LANES), jnp.float32)),
        mesh=mesh,
        scratch_types=[
            pltpu.VMEM((TPW, D), jnp.float32),
            pltpu.VMEM((TPW,), jnp.int32),
            pltpu.VMEM((TPW,), jnp.int32),
            pltpu.VMEM((TPW, LANES), jnp.float32),
            pltpu.VMEM((TPW, LANES), jnp.float32),
            pltpu.SemaphoreType.DMA,
        ],
    )(x, p0, p1, w0r, w1r)


def _group_ffn_body(be_ref, bv_ref, xs_hbm, ws_hbm, w1_hbm, b1_hbm, w3_hbm,
                    b3_hbm, w2_hbm, b2_hbm, o_hbm):
    def inner(xs_ref, ws_ref, w1_ref, b1_ref, w3_ref, b3_ref, w2_ref, b2_ref,
              o_ref):
        i = pl.program_id(0)

        @pl.when(bv_ref[i] > 0)
        def _():
            x = xs_ref[...]
            h1 = jnp.dot(x, w1_ref[0], preferred_element_type=jnp.float32)
            h1 = h1 + b1_ref[0]
            h3 = jnp.dot(x, w3_ref[0], preferred_element_type=jnp.float32)
            h3 = h3 + b3_ref[0]
            hh = (h1 * jax.nn.sigmoid(h1)) * h3
            o = jnp.dot(hh, w2_ref[0], preferred_element_type=jnp.float32)
            o_ref[...] = (o + b2_ref[0]) * ws_ref[:, :1]

    la = lambda: pl.Buffered(2, use_lookahead=True)
    pltpu.emit_pipeline(
        inner,
        grid=(NBR,),
        in_specs=[
            pl.BlockSpec((BT, D), lambda i: (i, 0)),
            pl.BlockSpec((BT, LANES), lambda i: (i, 0)),
            pl.BlockSpec((1, D, F), lambda i: (be_ref[i], 0, 0),
                         pipeline_mode=la()),
            pl.BlockSpec((1, 1, F), lambda i: (be_ref[i], 0, 0),
                         pipeline_mode=la()),
            pl.BlockSpec((1, D, F), lambda i: (be_ref[i], 0, 0),
                         pipeline_mode=la()),
            pl.BlockSpec((1, 1, F), lambda i: (be_ref[i], 0, 0),
                         pipeline_mode=la()),
            pl.BlockSpec((1, F, D), lambda i: (be_ref[i], 0, 0),
                         pipeline_mode=la()),
            pl.BlockSpec((1, 1, D), lambda i: (be_ref[i], 0, 0),
                         pipeline_mode=la()),
        ],
        out_specs=[pl.BlockSpec((BT, D), lambda i: (i, 0))],
    )(xs_hbm, ws_hbm, w1_hbm, b1_hbm, w3_hbm, b3_hbm, w2_hbm, b2_hbm, o_hbm)


def _group_ffn(bearr, bvarr, xs, ws, ew1, eb1, ew3, eb3, ew2, eb2):
    grid_spec = pltpu.PrefetchScalarGridSpec(
        num_scalar_prefetch=2,
        grid=(1,),
        in_specs=[pl.BlockSpec(memory_space=pl.ANY)] * 8,
        out_specs=pl.BlockSpec(memory_space=pl.ANY),
    )
    return pl.pallas_call(
        _group_ffn_body,
        grid_spec=grid_spec,
        out_shape=jax.ShapeDtypeStruct((GR, D), jnp.float32),
        compiler_params=pltpu.CompilerParams(
            dimension_semantics=("arbitrary",)),
    )(bearr, bvarr, xs, ws, ew1, eb1.reshape(E, 1, F), ew3,
      eb3.reshape(E, 1, F), ew2, eb2.reshape(E, 1, D))


def _combine_body(os_hbm, z_hbm, p0_hbm, p1_hbm, y_hbm,
                  g0a, g1a, gza, g0b, g1b, gzb, yv, i0a, i1a, i0b, i1b,
                  sema, semb):
    wid = lax.axis_index("s") * NC + lax.axis_index("c")
    base = wid * TPW
    bufs = ((g0a, g1a, gza, i0a, i1a, sema),
            (g0b, g1b, gzb, i0b, i1b, semb))

    def issue(ch, g0, g1, gz, i0, i1, sem):
        tb = base + ch * DCH
        pltpu.sync_copy(p0_hbm.at[pl.ds(tb, DCH)], i0)
        pltpu.sync_copy(p1_hbm.at[pl.ds(tb, DCH)], i1)
        return (pltpu.async_copy(os_hbm.at[i0], g0, sem),
                pltpu.async_copy(os_hbm.at[i1], g1, sem),
                pltpu.async_copy(z_hbm.at[pl.ds(tb, DCH)], gz, sem))

    nch = TPW // DCH
    pend = issue(0, *bufs[0])
    for ch in range(nch):
        g0, g1, gz = bufs[ch % 2][:3]
        for cp in pend:
            cp.wait()
        if ch + 1 < nch:
            pend = issue(ch + 1, *bufs[(ch + 1) % 2])
        for t in range(DCH):

            def body(j, _):
                for u in range(4):
                    sl = pl.ds(j * 64 + u * 16, 16)
                    yv[t, sl] = g0[t, sl] + g1[t, sl] + gz[t, sl]
                return 0

            lax.fori_loop(0, D // 64, body, 0)
        pltpu.sync_copy(yv, y_hbm.at[pl.ds(base + ch * DCH, DCH)])


def _combine(outs, z, p0, p1):
    f32 = jnp.float32
    mesh = plsc.VectorSubcoreMesh(core_axis_name="c", subcore_axis_name="s")
    return pl.kernel(
        _combine_body,
        out_type=jax.ShapeDtypeStruct((T, D), f32),
        mesh=mesh,
        scratch_types=[
            pltpu.VMEM((DCH, D), f32),
            pltpu.VMEM((DCH, D), f32),
            pltpu.VMEM((DCH, D), f32),
            pltpu.VMEM((DCH, D), f32),
            pltpu.VMEM((DCH, D), f32),
            pltpu.VMEM((DCH, D), f32),
            pltpu.VMEM((DCH, D), f32),
            pltpu.VMEM((DCH,), jnp.int32),
            pltpu.VMEM((DCH,), jnp.int32),
            pltpu.VMEM((DCH,), jnp.int32),
            pltpu.VMEM((DCH,), jnp.int32),
            pltpu.SemaphoreType.DMA,
            pltpu.SemaphoreType.DMA,
        ],
    )(outs, z, p0, p1)


def kernel(x, gate_w, ew1, eb1, ew2, eb2, ew3, eb3, sw1, sb1, sw2, sb2,
           sw3, sb3):
    f32 = jnp.float32
    gwt = jnp.zeros((D, LANES), f32).at[:, :E].set(gate_w.T)
    z, a1, a2, w1t, w2t, rank, cnt = _route_shared(
        x, gwt, sw1, sb1.reshape(1, F), sw3, sb3.reshape(1, F),
        sw2, sb2.reshape(1, D))
    p0f, p1f, bem, bvm = _dispatch_meta(cnt, a1, a2, rank)
    p0 = p0f[:, 0]
    p1 = p1f[:, 0]
    bearr = bem[:NBR, 0]
    bvarr = bvm[:NBR, 0]
    xs, ws = _scatter_sorted(x, p0, p1, w1t, w2t)
    outs = _group_ffn(bearr, bvarr, xs, ws, ew1, eb1, ew3, eb3, ew2, eb2)
    y = _combine(outs, z, p0, p1)
    return y


# speedup vs baseline: 1.5294x; 1.0195x over previous
import functools

import jax
import jax.numpy as jnp
from jax import lax
from jax.experimental import pallas as pl
from jax.experimental.pallas import tpu as pltpu
from jax.experimental.pallas import tpu_sc as plsc

T, D, F, E, K = 2048, 1024, 2048, 8, 2
LANES = 128
TB = 256
NTB = T // TB
BT = 256
NBR = -(-(T * K + E * (BT - 1)) // BT)
GR = NBR * BT
NC, NS = 2, 16
NW = NC * NS
TPW = T // NW
DCH = 16

_HI = lax.Precision.HIGHEST


def _fiota(shape, dim):
    return lax.broadcasted_iota(jnp.int32, shape, dim).astype(jnp.float32)


def _route_body(x_ref, gwt_ref, a1_ref, a2_ref, w1o_ref, w2o_ref, rank_ref,
                cnt_ref, carry):
    pid = pl.program_id(0)

    @pl.when(pid == 0)
    def _():
        carry[...] = jnp.zeros_like(carry)

    x = x_ref[...]
    s = jnp.dot(x, gwt_ref[...], preferred_element_type=jnp.float32)
    lane = _fiota((TB, E), 1)
    mx = jnp.max(s, axis=1, keepdims=True)
    ex = jnp.exp(s - mx)
    p = ex / jnp.sum(ex, axis=1, keepdims=True)
    m1 = jnp.max(p, axis=1, keepdims=True)
    is1 = p == m1
    a1 = jnp.min(jnp.where(is1, lane, float(LANES)), axis=1, keepdims=True)
    not1 = lane != a1
    p2 = jnp.where(not1, p, -1.0)
    m2 = jnp.max(p2, axis=1, keepdims=True)
    is2 = jnp.logical_and(p2 == m2, not1)
    a2 = jnp.min(jnp.where(is2, lane, float(LANES)), axis=1, keepdims=True)

    ind = jnp.where(lane == a1, 1.0, 0.0) + jnp.where(lane == a2, 1.0, 0.0)
    row = _fiota((TB, TB), 0)
    col = _fiota((TB, TB), 1)
    ltri = jnp.where(row >= col, 1.0, 0.0)
    incl = lax.dot(ltri, ind, precision=_HI)
    rank_ref[...] = incl - ind + carry[...]
    carry[...] = carry[...] + jnp.sum(ind, axis=0, keepdims=True)
    cnt_ref[...] = carry[...]

    a1_ref[...] = jnp.broadcast_to(a1, (TB, E))
    a2_ref[...] = jnp.broadcast_to(a2, (TB, E))
    w1o_ref[...] = jnp.broadcast_to(m1, (TB, LANES))
    w2o_ref[...] = jnp.broadcast_to(m2, (TB, LANES))


def _route(x, gwt):
    f32 = jnp.float32
    out_shapes = (
        jax.ShapeDtypeStruct((T, E), f32),
        jax.ShapeDtypeStruct((T, E), f32),
        jax.ShapeDtypeStruct((T, LANES), f32),
        jax.ShapeDtypeStruct((T, LANES), f32),
        jax.ShapeDtypeStruct((T, E), f32),
        jax.ShapeDtypeStruct((1, E), f32),
    )
    blk = lambda i: (i, 0)
    whole = lambda i: (0, 0)
    return pl.pallas_call(
        _route_body,
        grid=(NTB,),
        in_specs=[
            pl.BlockSpec((TB, D), blk),
            pl.BlockSpec((D, E), whole),
        ],
        out_specs=(
            pl.BlockSpec((TB, E), blk),
            pl.BlockSpec((TB, E), blk),
            pl.BlockSpec((TB, LANES), blk),
            pl.BlockSpec((TB, LANES), blk),
            pl.BlockSpec((TB, E), blk),
            pl.BlockSpec((1, E), whole),
        ),
        out_shape=out_shapes,
        scratch_shapes=[pltpu.VMEM((1, E), f32)],
        compiler_params=pltpu.CompilerParams(
            dimension_semantics=("arbitrary",)),
    )(x, gwt)


def _shared_body(x_ref, sw1_ref, sb1_ref, sw3_ref, sb3_ref, sw2_ref, sb2_ref,
                 z_ref):
    x = x_ref[...]
    h1 = jnp.dot(x, sw1_ref[...], preferred_element_type=jnp.float32)
    h1 = h1 + sb1_ref[...]
    h3 = jnp.dot(x, sw3_ref[...], preferred_element_type=jnp.float32)
    h3 = h3 + sb3_ref[...]
    hh = (h1 * jax.nn.sigmoid(h1)) * h3
    z = jnp.dot(hh, sw2_ref[...], preferred_element_type=jnp.float32)
    z_ref[...] = z + sb2_ref[...]


def _shared_ffn(x, sw1, sb1, sw3, sb3, sw2, sb2):
    blk = lambda i: (i, 0)
    whole = lambda i: (0, 0)
    return pl.pallas_call(
        _shared_body,
        grid=(NTB,),
        in_specs=[
            pl.BlockSpec((TB, D), blk),
            pl.BlockSpec((D, F), whole),
            pl.BlockSpec((1, F), whole),
            pl.BlockSpec((D, F), whole),
            pl.BlockSpec((1, F), whole),
            pl.BlockSpec((F, D), whole),
            pl.BlockSpec((1, D), whole),
        ],
        out_specs=pl.BlockSpec((TB, D), blk),
        out_shape=jax.ShapeDtypeStruct((T, D), jnp.float32),
        compiler_params=pltpu.CompilerParams(
            dimension_semantics=("arbitrary",)),
    )(x, sw1, sb1, sw3, sb3, sw2, sb2)


def _dispatch_meta_body(cnt_ref, a1_ref, a2_ref, rank_ref,
                        p0_ref, p1_ref, bem_ref, bvm_ref):
    c = cnt_ref[...]
    rc = jnp.floor((c + float(BT - 1)) / float(BT)) * float(BT)
    ri = _fiota((E, E), 0)
    ci = _fiota((E, E), 1)
    ut = jnp.where(ri <= ci, 1.0, 0.0)
    cum_incl = lax.dot(rc, ut, precision=_HI)
    cum_excl = cum_incl - rc

    rank = rank_ref[...]
    lane = _fiota((TB, E), 1)
    tot = cum_excl + rank
    pos0 = jnp.sum(jnp.where(lane == a1_ref[...], tot, 0.0),
                   axis=1, keepdims=True)
    pos1 = jnp.sum(jnp.where(lane == a2_ref[...], tot, 0.0),
                   axis=1, keepdims=True)
    p0_ref[...] = jnp.broadcast_to(pos0, (TB, E)).astype(jnp.int32)
    p1_ref[...] = jnp.broadcast_to(pos1, (TB, E)).astype(jnp.int32)

    @pl.when(pl.program_id(0) == 0)
    def _():
        bj = _fiota((LANES, E), 0)
        ce = _fiota((LANES, E), 1)
        start = bj * float(BT)
        endb = jnp.broadcast_to(cum_incl, (LANES, E))
        be = jnp.sum(jnp.where(start >= endb, 1.0, 0.0),
                     axis=1, keepdims=True)
        be = jnp.minimum(be, float(E - 1))
        bem_ref[...] = jnp.broadcast_to(be, (LANES, E)).astype(jnp.int32)
        exb = jnp.broadcast_to(cum_excl, (LANES, E))
        cb = jnp.broadcast_to(c, (LANES, E))
        has = jnp.logical_and(start >= exb, start < exb + cb)
        bv = jnp.sum(jnp.where(has, 1.0, 0.0), axis=1, keepdims=True)
        bvm_ref[...] = jnp.broadcast_to(bv, (LANES, E)).astype(jnp.int32)


def _dispatch_meta(cnt, a1, a2, rank):
    i32 = jnp.int32
    blk = lambda i: (i, 0)
    whole = lambda i: (0, 0)
    return pl.pallas_call(
        _dispatch_meta_body,
        grid=(NTB,),
        in_specs=[
            pl.BlockSpec((1, E), whole),
            pl.BlockSpec((TB, E), blk),
            pl.BlockSpec((TB, E), blk),
            pl.BlockSpec((TB, E), blk),
        ],
        out_specs=(
            pl.BlockSpec((TB, E), blk),
            pl.BlockSpec((TB, E), blk),
            pl.BlockSpec((LANES, E), whole),
            pl.BlockSpec((LANES, E), whole),
        ),
        out_shape=(
            jax.ShapeDtypeStruct((T, E), i32),
            jax.ShapeDtypeStruct((T, E), i32),
            jax.ShapeDtypeStruct((LANES, E), i32),
            jax.ShapeDtypeStruct((LANES, E), i32),
        ),
        compiler_params=pltpu.CompilerParams(
            dimension_semantics=("arbitrary",)),
    )(cnt, a1, a2, rank)


def _scatter_body(x_hbm, p0_hbm, p1_hbm, w0_hbm, w1_hbm, xs_hbm, ws_hbm,
                  xv, i0, i1, w0v, w1v, sem):
    wid = lax.axis_index("s") * NC + lax.axis_index("c")
    base = wid * TPW
    pltpu.sync_copy(x_hbm.at[pl.ds(base, TPW)], xv)
    pltpu.sync_copy(p0_hbm.at[pl.ds(base, TPW)], i0)
    pltpu.sync_copy(p1_hbm.at[pl.ds(base, TPW)], i1)
    pltpu.sync_copy(w0_hbm.at[pl.ds(base, TPW)], w0v)
    pltpu.sync_copy(w1_hbm.at[pl.ds(base, TPW)], w1v)
    a = pltpu.async_copy(xv, xs_hbm.at[i0], sem)
    b = pltpu.async_copy(xv, xs_hbm.at[i1], sem)
    c = pltpu.async_copy(w0v, ws_hbm.at[i0], sem)
    d = pltpu.async_copy(w1v, ws_hbm.at[i1], sem)
    a.wait()
    b.wait()
    c.wait()
    d.wait()


def _scatter_sorted(x, p0, p1, w0r, w1r):
    mesh = plsc.VectorSubcoreMesh(core_axis_name="c", subcore_axis_name="s")
    return pl.kernel(
        _scatter_body,
        out_type=(jax.ShapeDtypeStruct((GR, D), jnp.float32),
                  jax.ShapeDtypeStruct((GR, LANES), jnp.float32)),
        mesh=mesh,
        scratch_types=[
            pltpu.VMEM((TPW, D), jnp.float32),
            pltpu.VMEM((TPW,), jnp.int32),
            pltpu.VMEM((TPW,), jnp.int32),
            pltpu.VMEM((TPW, LANES), jnp.float32),
            pltpu.VMEM((TPW, LANES), jnp.float32),
            pltpu.SemaphoreType.DMA,
        ],
    )(x, p0, p1, w0r, w1r)


def _group_ffn_body(be_ref, bv_ref, xs_hbm, ws_hbm, w1_hbm, b1_hbm, w3_hbm,
                    b3_hbm, w2_hbm, b2_hbm, o_hbm):
    def inner(xs_ref, ws_ref, w1_ref, b1_ref, w3_ref, b3_ref, w2_ref, b2_ref,
              o_ref):
        i = pl.program_id(0)

        @pl.when(bv_ref[i] > 0)
        def _():
            x = xs_ref[...]
            h1 = jnp.dot(x, w1_ref[0], preferred_element_type=jnp.float32)
            h1 = h1 + b1_ref[0]
            h3 = jnp.dot(x, w3_ref[0], preferred_element_type=jnp.float32)
            h3 = h3 + b3_ref[0]
            hh = (h1 * jax.nn.sigmoid(h1)) * h3
            o = jnp.dot(hh, w2_ref[0], preferred_element_type=jnp.float32)
            o_ref[...] = (o + b2_ref[0]) * ws_ref[:, :1]

    la = lambda: pl.Buffered(2, use_lookahead=True)
    pltpu.emit_pipeline(
        inner,
        grid=(NBR,),
        in_specs=[
            pl.BlockSpec((BT, D), lambda i: (i, 0)),
            pl.BlockSpec((BT, LANES), lambda i: (i, 0)),
            pl.BlockSpec((1, D, F), lambda i: (be_ref[i], 0, 0),
                         pipeline_mode=la()),
            pl.BlockSpec((1, 1, F), lambda i: (be_ref[i], 0, 0),
                         pipeline_mode=la()),
            pl.BlockSpec((1, D, F), lambda i: (be_ref[i], 0, 0),
                         pipeline_mode=la()),
            pl.BlockSpec((1, 1, F), lambda i: (be_ref[i], 0, 0),
                         pipeline_mode=la()),
            pl.BlockSpec((1, F, D), lambda i: (be_ref[i], 0, 0),
                         pipeline_mode=la()),
            pl.BlockSpec((1, 1, D), lambda i: (be_ref[i], 0, 0),
                         pipeline_mode=la()),
        ],
        out_specs=[pl.BlockSpec((BT, D), lambda i: (i, 0))],
    )(xs_hbm, ws_hbm, w1_hbm, b1_hbm, w3_hbm, b3_hbm, w2_hbm, b2_hbm, o_hbm)


def _group_ffn(bearr, bvarr, xs, ws, ew1, eb1, ew3, eb3, ew2, eb2):
    grid_spec = pltpu.PrefetchScalarGridSpec(
        num_scalar_prefetch=2,
        grid=(1,),
        in_specs=[pl.BlockSpec(memory_space=pl.ANY)] * 8,
        out_specs=pl.BlockSpec(memory_space=pl.ANY),
    )
    return pl.pallas_call(
        _group_ffn_body,
        grid_spec=grid_spec,
        out_shape=jax.ShapeDtypeStruct((GR, D), jnp.float32),
        compiler_params=pltpu.CompilerParams(
            dimension_semantics=("arbitrary",)),
    )(bearr, bvarr, xs, ws, ew1, eb1.reshape(E, 1, F), ew3,
      eb3.reshape(E, 1, F), ew2, eb2.reshape(E, 1, D))


def _combine_body(os_hbm, z_hbm, p0_hbm, p1_hbm, y_hbm,
                  g0a, g1a, gza, g0b, g1b, gzb, yv, i0a, i1a, i0b, i1b,
                  sema, semb):
    wid = lax.axis_index("s") * NC + lax.axis_index("c")
    base = wid * TPW
    bufs = ((g0a, g1a, gza, i0a, i1a, sema),
            (g0b, g1b, gzb, i0b, i1b, semb))

    def issue(ch, g0, g1, gz, i0, i1, sem):
        tb = base + ch * DCH
        pltpu.sync_copy(p0_hbm.at[pl.ds(tb, DCH)], i0)
        pltpu.sync_copy(p1_hbm.at[pl.ds(tb, DCH)], i1)
        return (pltpu.async_copy(os_hbm.at[i0], g0, sem),
                pltpu.async_copy(os_hbm.at[i1], g1, sem),
                pltpu.async_copy(z_hbm.at[pl.ds(tb, DCH)], gz, sem))

    nch = TPW // DCH
    pend = issue(0, *bufs[0])
    for ch in range(nch):
        g0, g1, gz = bufs[ch % 2][:3]
        for cp in pend:
            cp.wait()
        if ch + 1 < nch:
            pend = issue(ch + 1, *bufs[(ch + 1) % 2])
        for t in range(DCH):

            def body(j, _):
                for u in range(4):
                    sl = pl.ds(j * 64 + u * 16, 16)
                    yv[t, sl] = g0[t, sl] + g1[t, sl] + gz[t, sl]
                return 0

            lax.fori_loop(0, D // 64, body, 0)
        pltpu.sync_copy(yv, y_hbm.at[pl.ds(base + ch * DCH, DCH)])


def _combine(outs, z, p0, p1):
    f32 = jnp.float32
    mesh = plsc.VectorSubcoreMesh(core_axis_name="c", subcore_axis_name="s")
    return pl.kernel(
        _combine_body,
        out_type=jax.ShapeDtypeStruct((T, D), f32),
        mesh=mesh,
        scratch_types=[
            pltpu.VMEM((DCH, D), f32),
            pltpu.VMEM((DCH, D), f32),
            pltpu.VMEM((DCH, D), f32),
            pltpu.VMEM((DCH, D), f32),
            pltpu.VMEM((DCH, D), f32),
            pltpu.VMEM((DCH, D), f32),
            pltpu.VMEM((DCH, D), f32),
            pltpu.VMEM((DCH,), jnp.int32),
            pltpu.VMEM((DCH,), jnp.int32),
            pltpu.VMEM((DCH,), jnp.int32),
            pltpu.VMEM((DCH,), jnp.int32),
            pltpu.SemaphoreType.DMA,
            pltpu.SemaphoreType.DMA,
        ],
    )(outs, z, p0, p1)


def kernel(x, gate_w, ew1, eb1, ew2, eb2, ew3, eb3, sw1, sb1, sw2, sb2,
           sw3, sb3):
    a1, a2, w1t, w2t, rank, cnt = _route(x, gate_w.T)
    p0f, p1f, bem, bvm = _dispatch_meta(cnt, a1, a2, rank)
    p0 = p0f[:, 0]
    p1 = p1f[:, 0]
    bearr = bem[:NBR, 0]
    bvarr = bvm[:NBR, 0]
    xs, ws = _scatter_sorted(x, p0, p1, w1t, w2t)
    z = _shared_ffn(x, sw1, sb1.reshape(1, F), sw3, sb3.reshape(1, F),
                    sw2, sb2.reshape(1, D))
    outs = _group_ffn(bearr, bvarr, xs, ws, ew1, eb1, ew3, eb3, ew2, eb2)
    y = _combine(outs, z, p0, p1)
    return y


# speedup vs baseline: 1.5701x; 1.0266x over previous
import functools

import jax
import jax.numpy as jnp
from jax import lax
from jax.experimental import pallas as pl
from jax.experimental.pallas import tpu as pltpu
from jax.experimental.pallas import tpu_sc as plsc

T, D, F, E, K = 2048, 1024, 2048, 8, 2
LANES = 128
TB = 256
NTB = T // TB
BT = 256
NBR = -(-(T * K + E * (BT - 1)) // BT)
GR = NBR * BT
NC, NS = 2, 16
NW = NC * NS
TPW = T // NW
DCH = 16

_HI = lax.Precision.HIGHEST


def _fiota(shape, dim):
    return lax.broadcasted_iota(jnp.int32, shape, dim).astype(jnp.float32)


def _route_body(x_ref, gwt_ref, a1_ref, a2_ref, w1o_ref, w2o_ref, rank_ref,
                cnt_ref, carry):
    pid = pl.program_id(0)

    @pl.when(pid == 0)
    def _():
        carry[...] = jnp.zeros_like(carry)

    x = x_ref[...]
    s = lax.dot_general(x, gwt_ref[...], (((1,), (1,)), ((), ())),
                        preferred_element_type=jnp.float32)
    lane = _fiota((TB, E), 1)
    mx = jnp.max(s, axis=1, keepdims=True)
    ex = jnp.exp(s - mx)
    p = ex / jnp.sum(ex, axis=1, keepdims=True)
    m1 = jnp.max(p, axis=1, keepdims=True)
    is1 = p == m1
    a1 = jnp.min(jnp.where(is1, lane, float(LANES)), axis=1, keepdims=True)
    not1 = lane != a1
    p2 = jnp.where(not1, p, -1.0)
    m2 = jnp.max(p2, axis=1, keepdims=True)
    is2 = jnp.logical_and(p2 == m2, not1)
    a2 = jnp.min(jnp.where(is2, lane, float(LANES)), axis=1, keepdims=True)

    ind = jnp.where(lane == a1, 1.0, 0.0) + jnp.where(lane == a2, 1.0, 0.0)
    row = _fiota((TB, TB), 0)
    col = _fiota((TB, TB), 1)
    ltri = jnp.where(row >= col, 1.0, 0.0)
    incl = lax.dot(ltri, ind)
    rank_ref[...] = incl - ind + carry[...]
    carry[...] = carry[...] + jnp.sum(ind, axis=0, keepdims=True)
    cnt_ref[...] = carry[...]

    a1_ref[...] = jnp.broadcast_to(a1, (TB, E))
    a2_ref[...] = jnp.broadcast_to(a2, (TB, E))
    w1o_ref[...] = jnp.broadcast_to(m1, (TB, LANES))
    w2o_ref[...] = jnp.broadcast_to(m2, (TB, LANES))


def _route(x, gwt):
    f32 = jnp.float32
    out_shapes = (
        jax.ShapeDtypeStruct((T, E), f32),
        jax.ShapeDtypeStruct((T, E), f32),
        jax.ShapeDtypeStruct((T, LANES), f32),
        jax.ShapeDtypeStruct((T, LANES), f32),
        jax.ShapeDtypeStruct((T, E), f32),
        jax.ShapeDtypeStruct((1, E), f32),
    )
    blk = lambda i: (i, 0)
    whole = lambda i: (0, 0)
    return pl.pallas_call(
        _route_body,
        grid=(NTB,),
        in_specs=[
            pl.BlockSpec((TB, D), blk),
            pl.BlockSpec((E, D), whole),
        ],
        out_specs=(
            pl.BlockSpec((TB, E), blk),
            pl.BlockSpec((TB, E), blk),
            pl.BlockSpec((TB, LANES), blk),
            pl.BlockSpec((TB, LANES), blk),
            pl.BlockSpec((TB, E), blk),
            pl.BlockSpec((1, E), whole),
        ),
        out_shape=out_shapes,
        scratch_shapes=[pltpu.VMEM((1, E), f32)],
        compiler_params=pltpu.CompilerParams(
            dimension_semantics=("arbitrary",)),
    )(x, gwt)


def _shared_body(x_ref, sw1_ref, sb1_ref, sw3_ref, sb3_ref, sw2_ref, sb2_ref,
                 z_ref):
    x = x_ref[...]
    h1 = jnp.dot(x, sw1_ref[...], preferred_element_type=jnp.float32)
    h1 = h1 + sb1_ref[...]
    h3 = jnp.dot(x, sw3_ref[...], preferred_element_type=jnp.float32)
    h3 = h3 + sb3_ref[...]
    hh = (h1 * jax.nn.sigmoid(h1)) * h3
    z = jnp.dot(hh, sw2_ref[...], preferred_element_type=jnp.float32)
    z_ref[...] = z + sb2_ref[...]


def _shared_ffn(x, sw1, sb1, sw3, sb3, sw2, sb2):
    blk = lambda i: (i, 0)
    whole = lambda i: (0, 0)
    return pl.pallas_call(
        _shared_body,
        grid=(NTB,),
        in_specs=[
            pl.BlockSpec((TB, D), blk),
            pl.BlockSpec((D, F), whole),
            pl.BlockSpec((1, F), whole),
            pl.BlockSpec((D, F), whole),
            pl.BlockSpec((1, F), whole),
            pl.BlockSpec((F, D), whole),
            pl.BlockSpec((1, D), whole),
        ],
        out_specs=pl.BlockSpec((TB, D), blk),
        out_shape=jax.ShapeDtypeStruct((T, D), jnp.float32),
        compiler_params=pltpu.CompilerParams(
            dimension_semantics=("arbitrary",)),
    )(x, sw1, sb1, sw3, sb3, sw2, sb2)


def _dispatch_meta_body(cnt_ref, a1_ref, a2_ref, rank_ref,
                        p0_ref, p1_ref, bem_ref, bvm_ref):
    c = cnt_ref[...]
    rc = jnp.floor((c + float(BT - 1)) / float(BT)) * float(BT)
    ri = _fiota((E, E), 0)
    ci = _fiota((E, E), 1)
    ut = jnp.where(ri <= ci, 1.0, 0.0)
    cum_incl = lax.dot(rc, ut)
    cum_excl = cum_incl - rc

    rank = rank_ref[...]
    lane = _fiota((TB, E), 1)
    tot = cum_excl + rank
    pos0 = jnp.sum(jnp.where(lane == a1_ref[...], tot, 0.0),
                   axis=1, keepdims=True)
    pos1 = jnp.sum(jnp.where(lane == a2_ref[...], tot, 0.0),
                   axis=1, keepdims=True)
    p0_ref[...] = jnp.broadcast_to(pos0, (TB, E)).astype(jnp.int32)
    p1_ref[...] = jnp.broadcast_to(pos1, (TB, E)).astype(jnp.int32)

    @pl.when(pl.program_id(0) == 0)
    def _():
        bj = _fiota((LANES, E), 0)
        ce = _fiota((LANES, E), 1)
        start = bj * float(BT)
        endb = jnp.broadcast_to(cum_incl, (LANES, E))
        be = jnp.sum(jnp.where(start >= endb, 1.0, 0.0),
                     axis=1, keepdims=True)
        be = jnp.minimum(be, float(E - 1))
        bem_ref[...] = jnp.broadcast_to(be, (LANES, E)).astype(jnp.int32)
        exb = jnp.broadcast_to(cum_excl, (LANES, E))
        cb = jnp.broadcast_to(c, (LANES, E))
        has = jnp.logical_and(start >= exb, start < exb + cb)
        bv = jnp.sum(jnp.where(has, 1.0, 0.0), axis=1, keepdims=True)
        bvm_ref[...] = jnp.broadcast_to(bv, (LANES, E)).astype(jnp.int32)


def _dispatch_meta(cnt, a1, a2, rank):
    i32 = jnp.int32
    blk = lambda i: (i, 0)
    whole = lambda i: (0, 0)
    return pl.pallas_call(
        _dispatch_meta_body,
        grid=(NTB,),
        in_specs=[
            pl.BlockSpec((1, E), whole),
            pl.BlockSpec((TB, E), blk),
            pl.BlockSpec((TB, E), blk),
            pl.BlockSpec((TB, E), blk),
        ],
        out_specs=(
            pl.BlockSpec((TB, E), blk),
            pl.BlockSpec((TB, E), blk),
            pl.BlockSpec((LANES, E), whole),
            pl.BlockSpec((LANES, E), whole),
        ),
        out_shape=(
            jax.ShapeDtypeStruct((T, E), i32),
            jax.ShapeDtypeStruct((T, E), i32),
            jax.ShapeDtypeStruct((LANES, E), i32),
            jax.ShapeDtypeStruct((LANES, E), i32),
        ),
        compiler_params=pltpu.CompilerParams(
            dimension_semantics=("arbitrary",)),
    )(cnt, a1, a2, rank)


def _scatter_body(x_hbm, p0_hbm, p1_hbm, w0_hbm, w1_hbm, xs_hbm, ws_hbm,
                  xv, i0, i1, w0v, w1v, sem):
    wid = lax.axis_index("s") * NC + lax.axis_index("c")
    base = wid * TPW
    pltpu.sync_copy(x_hbm.at[pl.ds(base, TPW)], xv)
    pltpu.sync_copy(p0_hbm.at[pl.ds(base, TPW)], i0)
    pltpu.sync_copy(p1_hbm.at[pl.ds(base, TPW)], i1)
    pltpu.sync_copy(w0_hbm.at[pl.ds(base, TPW)], w0v)
    pltpu.sync_copy(w1_hbm.at[pl.ds(base, TPW)], w1v)
    a = pltpu.async_copy(xv, xs_hbm.at[i0], sem)
    b = pltpu.async_copy(xv, xs_hbm.at[i1], sem)
    c = pltpu.async_copy(w0v, ws_hbm.at[i0], sem)
    d = pltpu.async_copy(w1v, ws_hbm.at[i1], sem)
    a.wait()
    b.wait()
    c.wait()
    d.wait()


def _scatter_sorted(x, p0, p1, w0r, w1r):
    mesh = plsc.VectorSubcoreMesh(core_axis_name="c", subcore_axis_name="s")
    return pl.kernel(
        _scatter_body,
        out_type=(jax.ShapeDtypeStruct((GR, D), jnp.float32),
                  jax.ShapeDtypeStruct((GR, LANES), jnp.float32)),
        mesh=mesh,
        scratch_types=[
            pltpu.VMEM((TPW, D), jnp.float32),
            pltpu.VMEM((TPW,), jnp.int32),
            pltpu.VMEM((TPW,), jnp.int32),
            pltpu.VMEM((TPW, LANES), jnp.float32),
            pltpu.VMEM((TPW, LANES), jnp.float32),
            pltpu.SemaphoreType.DMA,
        ],
    )(x, p0, p1, w0r, w1r)


def _group_ffn_body(be_ref, bv_ref, xs_hbm, ws_hbm, w1_hbm, b1_hbm, w3_hbm,
                    b3_hbm, w2_hbm, b2_hbm, o_hbm):
    def inner(xs_ref, ws_ref, w1_ref, b1_ref, w3_ref, b3_ref, w2_ref, b2_ref,
              o_ref):
        i = pl.program_id(0)

        @pl.when(bv_ref[i] > 0)
        def _():
            x = xs_ref[...]
            h1 = jnp.dot(x, w1_ref[0], preferred_element_type=jnp.float32)
            h1 = h1 + b1_ref[0]
            h3 = jnp.dot(x, w3_ref[0], preferred_element_type=jnp.float32)
            h3 = h3 + b3_ref[0]
            hh = (h1 * jax.nn.sigmoid(h1)) * h3
            o = jnp.dot(hh, w2_ref[0], preferred_element_type=jnp.float32)
            o_ref[...] = (o + b2_ref[0]) * ws_ref[:, :1]

    la = lambda: pl.Buffered(2, use_lookahead=True)
    pltpu.emit_pipeline(
        inner,
        grid=(NBR,),
        in_specs=[
            pl.BlockSpec((BT, D), lambda i: (i, 0)),
            pl.BlockSpec((BT, LANES), lambda i: (i, 0)),
            pl.BlockSpec((1, D, F), lambda i: (be_ref[i], 0, 0),
                         pipeline_mode=la()),
            pl.BlockSpec((1, 1, F), lambda i: (be_ref[i], 0, 0),
                         pipeline_mode=la()),
            pl.BlockSpec((1, D, F), lambda i: (be_ref[i], 0, 0),
                         pipeline_mode=la()),
            pl.BlockSpec((1, 1, F), lambda i: (be_ref[i], 0, 0),
                         pipeline_mode=la()),
            pl.BlockSpec((1, F, D), lambda i: (be_ref[i], 0, 0),
                         pipeline_mode=la()),
            pl.BlockSpec((1, 1, D), lambda i: (be_ref[i], 0, 0),
                         pipeline_mode=la()),
        ],
        out_specs=[pl.BlockSpec((BT, D), lambda i: (i, 0))],
    )(xs_hbm, ws_hbm, w1_hbm, b1_hbm, w3_hbm, b3_hbm, w2_hbm, b2_hbm, o_hbm)


def _group_ffn(bearr, bvarr, xs, ws, ew1, eb1, ew3, eb3, ew2, eb2):
    grid_spec = pltpu.PrefetchScalarGridSpec(
        num_scalar_prefetch=2,
        grid=(1,),
        in_specs=[pl.BlockSpec(memory_space=pl.ANY)] * 8,
        out_specs=pl.BlockSpec(memory_space=pl.ANY),
    )
    return pl.pallas_call(
        _group_ffn_body,
        grid_spec=grid_spec,
        out_shape=jax.ShapeDtypeStruct((GR, D), jnp.float32),
        compiler_params=pltpu.CompilerParams(
            dimension_semantics=("arbitrary",)),
    )(bearr, bvarr, xs, ws, ew1, eb1.reshape(E, 1, F), ew3,
      eb3.reshape(E, 1, F), ew2, eb2.reshape(E, 1, D))


def _combine_body(os_hbm, z_hbm, p0_hbm, p1_hbm, y_hbm,
                  g0a, g1a, gza, g0b, g1b, gzb, i0a, i1a, i0b, i1b,
                  sema, semb, wsa, wsb):
    wid = lax.axis_index("s") * NC + lax.axis_index("c")
    base = wid * TPW
    bufs = ((g0a, g1a, gza, i0a, i1a, sema),
            (g0b, g1b, gzb, i0b, i1b, semb))

    def issue(ch, g0, g1, gz, i0, i1, sem):
        tb = base + ch * DCH
        pltpu.sync_copy(p0_hbm.at[pl.ds(tb, DCH)], i0)
        pltpu.sync_copy(p1_hbm.at[pl.ds(tb, DCH)], i1)
        return (pltpu.async_copy(os_hbm.at[i0], g0, sem),
                pltpu.async_copy(os_hbm.at[i1], g1, sem),
                pltpu.async_copy(z_hbm.at[pl.ds(tb, DCH)], gz, sem))

    nch = TPW // DCH
    wsems = (wsa, wsb)
    pend = issue(0, *bufs[0])
    wb = [None, None]
    for ch in range(nch):
        g0, g1, gz = bufs[ch % 2][:3]
        for cp in pend:
            cp.wait()
        if ch + 1 < nch:
            if wb[(ch + 1) % 2] is not None:
                wb[(ch + 1) % 2].wait()
                wb[(ch + 1) % 2] = None
            pend = issue(ch + 1, *bufs[(ch + 1) % 2])
        for t in range(DCH):

            def body(j, _):
                for u in range(4):
                    sl = pl.ds(j * 64 + u * 16, 16)
                    gz[t, sl] = gz[t, sl] + g0[t, sl] + g1[t, sl]
                return 0

            lax.fori_loop(0, D // 64, body, 0)
        wb[ch % 2] = pltpu.async_copy(
            gz, y_hbm.at[pl.ds(base + ch * DCH, DCH)], wsems[ch % 2])
    for w in wb:
        if w is not None:
            w.wait()


def _combine(outs, z, p0, p1):
    f32 = jnp.float32
    mesh = plsc.VectorSubcoreMesh(core_axis_name="c", subcore_axis_name="s")
    return pl.kernel(
        _combine_body,
        out_type=jax.ShapeDtypeStruct((T, D), f32),
        mesh=mesh,
        scratch_types=[
            pltpu.VMEM((DCH, D), f32),
            pltpu.VMEM((DCH, D), f32),
            pltpu.VMEM((DCH, D), f32),
            pltpu.VMEM((DCH, D), f32),
            pltpu.VMEM((DCH, D), f32),
            pltpu.VMEM((DCH, D), f32),
            pltpu.VMEM((DCH,), jnp.int32),
            pltpu.VMEM((DCH,), jnp.int32),
            pltpu.VMEM((DCH,), jnp.int32),
            pltpu.VMEM((DCH,), jnp.int32),
            pltpu.SemaphoreType.DMA,
            pltpu.SemaphoreType.DMA,
            pltpu.SemaphoreType.DMA,
            pltpu.SemaphoreType.DMA,
        ],
    )(outs, z, p0, p1)


def kernel(x, gate_w, ew1, eb1, ew2, eb2, ew3, eb3, sw1, sb1, sw2, sb2,
           sw3, sb3):
    a1, a2, w1t, w2t, rank, cnt = _route(x, gate_w)
    p0f, p1f, bem, bvm = _dispatch_meta(cnt, a1, a2, rank)
    p0 = p0f[:, 0]
    p1 = p1f[:, 0]
    bearr = bem[:NBR, 0]
    bvarr = bvm[:NBR, 0]
    xs, ws = _scatter_sorted(x, p0, p1, w1t, w2t)
    z = _shared_ffn(x, sw1, sb1.reshape(1, F), sw3, sb3.reshape(1, F),
                    sw2, sb2.reshape(1, D))
    outs = _group_ffn(bearr, bvarr, xs, ws, ew1, eb1, ew3, eb3, ew2, eb2)
    y = _combine(outs, z, p0, p1)
    return y


# speedup vs baseline: 1.6048x; 1.0221x over previous
import functools

import jax
import jax.numpy as jnp
from jax import lax
from jax.experimental import pallas as pl
from jax.experimental.pallas import tpu as pltpu
from jax.experimental.pallas import tpu_sc as plsc

T, D, F, E, K = 2048, 1024, 2048, 8, 2
LANES = 128
TB = 256
NTB = T // TB
BT = 256
NBR = -(-(T * K + E * (BT - 1)) // BT)
GR = NBR * BT
NC, NS = 2, 16
NW = NC * NS
TPW = T // NW
DCH = 16

_HI = lax.Precision.HIGHEST


def _fiota(shape, dim):
    return lax.broadcasted_iota(jnp.int32, shape, dim).astype(jnp.float32)


def _route_body(x_ref, gwt_ref, a1_ref, a2_ref, w1o_ref, w2o_ref, rank_ref,
                cnt_ref, carry):
    pid = pl.program_id(0)

    @pl.when(pid == 0)
    def _():
        carry[...] = jnp.zeros_like(carry)

    x = x_ref[...]
    s = lax.dot_general(x, gwt_ref[...], (((1,), (1,)), ((), ())),
                        preferred_element_type=jnp.float32)
    lane = _fiota((TB, E), 1)
    mx = jnp.max(s, axis=1, keepdims=True)
    ex = jnp.exp(s - mx)
    p = ex / jnp.sum(ex, axis=1, keepdims=True)
    m1 = jnp.max(p, axis=1, keepdims=True)
    is1 = p == m1
    a1 = jnp.min(jnp.where(is1, lane, float(LANES)), axis=1, keepdims=True)
    not1 = lane != a1
    p2 = jnp.where(not1, p, -1.0)
    m2 = jnp.max(p2, axis=1, keepdims=True)
    is2 = jnp.logical_and(p2 == m2, not1)
    a2 = jnp.min(jnp.where(is2, lane, float(LANES)), axis=1, keepdims=True)

    ind = jnp.where(lane == a1, 1.0, 0.0) + jnp.where(lane == a2, 1.0, 0.0)
    row = _fiota((TB, TB), 0)
    col = _fiota((TB, TB), 1)
    ltri = jnp.where(row >= col, 1.0, 0.0)
    incl = lax.dot(ltri, ind)
    rank_ref[...] = incl - ind + carry[...]
    carry[...] = carry[...] + jnp.sum(ind, axis=0, keepdims=True)
    cnt_ref[...] = carry[...]

    a1_ref[...] = jnp.broadcast_to(a1, (TB, E))
    a2_ref[...] = jnp.broadcast_to(a2, (TB, E))
    w1o_ref[...] = jnp.broadcast_to(m1, (TB, LANES))
    w2o_ref[...] = jnp.broadcast_to(m2, (TB, LANES))


def _route(x, gwt):
    f32 = jnp.float32
    out_shapes = (
        jax.ShapeDtypeStruct((T, E), f32),
        jax.ShapeDtypeStruct((T, E), f32),
        jax.ShapeDtypeStruct((T, LANES), f32),
        jax.ShapeDtypeStruct((T, LANES), f32),
        jax.ShapeDtypeStruct((T, E), f32),
        jax.ShapeDtypeStruct((1, E), f32),
    )
    blk = lambda i: (i, 0)
    whole = lambda i: (0, 0)
    return pl.pallas_call(
        _route_body,
        grid=(NTB,),
        in_specs=[
            pl.BlockSpec((TB, D), blk),
            pl.BlockSpec((E, D), whole),
        ],
        out_specs=(
            pl.BlockSpec((TB, E), blk),
            pl.BlockSpec((TB, E), blk),
            pl.BlockSpec((TB, LANES), blk),
            pl.BlockSpec((TB, LANES), blk),
            pl.BlockSpec((TB, E), blk),
            pl.BlockSpec((1, E), whole),
        ),
        out_shape=out_shapes,
        scratch_shapes=[pltpu.VMEM((1, E), f32)],
        compiler_params=pltpu.CompilerParams(
            dimension_semantics=("arbitrary",)),
    )(x, gwt)


def _shared_body(x_ref, sw1_ref, sb1_ref, sw3_ref, sb3_ref, sw2_ref, sb2_ref,
                 z_ref):
    x = x_ref[...]
    h1 = jnp.dot(x, sw1_ref[...], preferred_element_type=jnp.float32)
    h1 = h1 + sb1_ref[...]
    h3 = jnp.dot(x, sw3_ref[...], preferred_element_type=jnp.float32)
    h3 = h3 + sb3_ref[...]
    hh = (h1 * jax.nn.sigmoid(h1)) * h3
    z = jnp.dot(hh, sw2_ref[...], preferred_element_type=jnp.float32)
    z_ref[...] = z + sb2_ref[...]


def _shared_ffn(x, sw1, sb1, sw3, sb3, sw2, sb2):
    blk = lambda i: (i, 0)
    whole = lambda i: (0, 0)
    return pl.pallas_call(
        _shared_body,
        grid=(NTB,),
        in_specs=[
            pl.BlockSpec((TB, D), blk),
            pl.BlockSpec((D, F), whole),
            pl.BlockSpec((1, F), whole),
            pl.BlockSpec((D, F), whole),
            pl.BlockSpec((1, F), whole),
            pl.BlockSpec((F, D), whole),
            pl.BlockSpec((1, D), whole),
        ],
        out_specs=pl.BlockSpec((TB, D), blk),
        out_shape=jax.ShapeDtypeStruct((T, D), jnp.float32),
        compiler_params=pltpu.CompilerParams(
            dimension_semantics=("arbitrary",)),
    )(x, sw1, sb1, sw3, sb3, sw2, sb2)


def _dispatch_meta_body(cnt_ref, a1_ref, a2_ref, rank_ref,
                        p0_ref, p1_ref, bem_ref, bvm_ref):
    c = cnt_ref[...]
    rc = jnp.floor((c + float(BT - 1)) / float(BT)) * float(BT)
    ri = _fiota((E, E), 0)
    ci = _fiota((E, E), 1)
    ut = jnp.where(ri <= ci, 1.0, 0.0)
    cum_incl = lax.dot(rc, ut)
    cum_excl = cum_incl - rc

    rank = rank_ref[...]
    lane = _fiota((TB, E), 1)
    tot = cum_excl + rank
    pos0 = jnp.sum(jnp.where(lane == a1_ref[...], tot, 0.0),
                   axis=1, keepdims=True)
    pos1 = jnp.sum(jnp.where(lane == a2_ref[...], tot, 0.0),
                   axis=1, keepdims=True)
    p0_ref[...] = jnp.broadcast_to(pos0, (TB, E)).astype(jnp.int32)
    p1_ref[...] = jnp.broadcast_to(pos1, (TB, E)).astype(jnp.int32)

    @pl.when(pl.program_id(0) == 0)
    def _():
        bj = _fiota((LANES, E), 0)
        ce = _fiota((LANES, E), 1)
        start = bj * float(BT)
        endb = jnp.broadcast_to(cum_incl, (LANES, E))
        be = jnp.sum(jnp.where(start >= endb, 1.0, 0.0),
                     axis=1, keepdims=True)
        be = jnp.minimum(be, float(E - 1))
        bem_ref[...] = jnp.broadcast_to(be, (LANES, E)).astype(jnp.int32)
        exb = jnp.broadcast_to(cum_excl, (LANES, E))
        cb = jnp.broadcast_to(c, (LANES, E))
        has = jnp.logical_and(start >= exb, start < exb + cb)
        bv = jnp.sum(jnp.where(has, 1.0, 0.0), axis=1, keepdims=True)
        bvm_ref[...] = jnp.broadcast_to(bv, (LANES, E)).astype(jnp.int32)


def _dispatch_meta(cnt, a1, a2, rank):
    i32 = jnp.int32
    blk = lambda i: (i, 0)
    whole = lambda i: (0, 0)
    return pl.pallas_call(
        _dispatch_meta_body,
        grid=(NTB,),
        in_specs=[
            pl.BlockSpec((1, E), whole),
            pl.BlockSpec((TB, E), blk),
            pl.BlockSpec((TB, E), blk),
            pl.BlockSpec((TB, E), blk),
        ],
        out_specs=(
            pl.BlockSpec((TB, E), blk),
            pl.BlockSpec((TB, E), blk),
            pl.BlockSpec((LANES, E), whole),
            pl.BlockSpec((LANES, E), whole),
        ),
        out_shape=(
            jax.ShapeDtypeStruct((T, E), i32),
            jax.ShapeDtypeStruct((T, E), i32),
            jax.ShapeDtypeStruct((LANES, E), i32),
            jax.ShapeDtypeStruct((LANES, E), i32),
        ),
        compiler_params=pltpu.CompilerParams(
            dimension_semantics=("arbitrary",)),
    )(cnt, a1, a2, rank)


def _scatter_body(x_hbm, p0_hbm, p1_hbm, w0_hbm, w1_hbm, xs_hbm, ws_hbm,
                  xv, i0, i1, w0v, w1v, sem):
    wid = lax.axis_index("s") * NC + lax.axis_index("c")
    base = wid * TPW
    pltpu.sync_copy(x_hbm.at[pl.ds(base, TPW)], xv)
    pltpu.sync_copy(p0_hbm.at[pl.ds(base, TPW)], i0)
    pltpu.sync_copy(p1_hbm.at[pl.ds(base, TPW)], i1)
    pltpu.sync_copy(w0_hbm.at[pl.ds(base, TPW)], w0v)
    pltpu.sync_copy(w1_hbm.at[pl.ds(base, TPW)], w1v)
    a = pltpu.async_copy(xv, xs_hbm.at[i0], sem)
    b = pltpu.async_copy(xv, xs_hbm.at[i1], sem)
    c = pltpu.async_copy(w0v, ws_hbm.at[i0], sem)
    d = pltpu.async_copy(w1v, ws_hbm.at[i1], sem)
    a.wait()
    b.wait()
    c.wait()
    d.wait()


def _scatter_sorted(x, p0, p1, w0r, w1r):
    mesh = plsc.VectorSubcoreMesh(core_axis_name="c", subcore_axis_name="s")
    return pl.kernel(
        _scatter_body,
        out_type=(jax.ShapeDtypeStruct((GR, D), jnp.float32),
                  jax.ShapeDtypeStruct((GR, LANES), jnp.float32)),
        mesh=mesh,
        scratch_types=[
            pltpu.VMEM((TPW, D), jnp.float32),
            pltpu.VMEM((TPW,), jnp.int32),
            pltpu.VMEM((TPW,), jnp.int32),
            pltpu.VMEM((TPW, LANES), jnp.float32),
            pltpu.VMEM((TPW, LANES), jnp.float32),
            pltpu.SemaphoreType.DMA,
        ],
    )(x, p0, p1, w0r, w1r)


def _group_ffn_body(be_ref, bv_ref, xs_hbm, ws_hbm, w1_hbm, b1_hbm, w3_hbm,
                    b3_hbm, w2_hbm, b2_hbm, o_hbm):
    def inner(xs_ref, ws_ref, w1_ref, b1_ref, w3_ref, b3_ref, w2_ref, b2_ref,
              o_ref):
        i = pl.program_id(0)

        @pl.when(bv_ref[i] > 0)
        def _():
            x = xs_ref[...]
            h1 = jnp.dot(x, w1_ref[0], preferred_element_type=jnp.float32)
            h1 = h1 + b1_ref[0]
            h3 = jnp.dot(x, w3_ref[0], preferred_element_type=jnp.float32)
            h3 = h3 + b3_ref[0]
            hh = (h1 * jax.nn.sigmoid(h1)) * h3
            o = jnp.dot(hh, w2_ref[0], preferred_element_type=jnp.float32)
            o_ref[...] = (o + b2_ref[0]) * ws_ref[:, :1]

    la = lambda: pl.Buffered(2, use_lookahead=True)
    pltpu.emit_pipeline(
        inner,
        grid=(NBR,),
        in_specs=[
            pl.BlockSpec((BT, D), lambda i: (i, 0)),
            pl.BlockSpec((BT, LANES), lambda i: (i, 0)),
            pl.BlockSpec((1, D, F), lambda i: (be_ref[i], 0, 0),
                         pipeline_mode=la()),
            pl.BlockSpec((1, 1, F), lambda i: (be_ref[i], 0, 0),
                         pipeline_mode=la()),
            pl.BlockSpec((1, D, F), lambda i: (be_ref[i], 0, 0),
                         pipeline_mode=la()),
            pl.BlockSpec((1, 1, F), lambda i: (be_ref[i], 0, 0),
                         pipeline_mode=la()),
            pl.BlockSpec((1, F, D), lambda i: (be_ref[i], 0, 0),
                         pipeline_mode=la()),
            pl.BlockSpec((1, 1, D), lambda i: (be_ref[i], 0, 0),
                         pipeline_mode=la()),
        ],
        out_specs=[pl.BlockSpec((BT, D), lambda i: (i, 0))],
    )(xs_hbm, ws_hbm, w1_hbm, b1_hbm, w3_hbm, b3_hbm, w2_hbm, b2_hbm, o_hbm)


def _group_ffn(bearr, bvarr, xs, ws, ew1, eb1, ew3, eb3, ew2, eb2):
    grid_spec = pltpu.PrefetchScalarGridSpec(
        num_scalar_prefetch=2,
        grid=(1,),
        in_specs=[pl.BlockSpec(memory_space=pl.ANY)] * 8,
        out_specs=pl.BlockSpec(memory_space=pl.ANY),
    )
    return pl.pallas_call(
        _group_ffn_body,
        grid_spec=grid_spec,
        out_shape=jax.ShapeDtypeStruct((GR, D), jnp.float32),
        compiler_params=pltpu.CompilerParams(
            dimension_semantics=("arbitrary",)),
    )(bearr, bvarr, xs, ws, ew1, eb1.reshape(E, 1, F), ew3,
      eb3.reshape(E, 1, F), ew2, eb2.reshape(E, 1, D))


def _combine_body(os_hbm, z_hbm, p0_hbm, p1_hbm, y_hbm,
                  g0a, g1a, gza, g0b, g1b, gzb, i0a, i1a, i0b, i1b,
                  sema, semb, wsa, wsb):
    wid = lax.axis_index("s") * NC + lax.axis_index("c")
    base = wid * TPW
    bufs = ((g0a, g1a, gza, i0a, i1a, sema),
            (g0b, g1b, gzb, i0b, i1b, semb))

    def issue(ch, g0, g1, gz, i0, i1, sem):
        tb = base + ch * DCH
        pltpu.sync_copy(p0_hbm.at[pl.ds(tb, DCH)], i0)
        pltpu.sync_copy(p1_hbm.at[pl.ds(tb, DCH)], i1)
        return (pltpu.async_copy(os_hbm.at[i0], g0, sem),
                pltpu.async_copy(os_hbm.at[i1], g1, sem),
                pltpu.async_copy(z_hbm.at[pl.ds(tb, DCH)], gz, sem))

    nch = TPW // DCH
    wsems = (wsa, wsb)
    pend = issue(0, *bufs[0])
    wb = [None, None]
    for ch in range(nch):
        g0, g1, gz = bufs[ch % 2][:3]
        for cp in pend:
            cp.wait()
        if ch + 1 < nch:
            if wb[(ch + 1) % 2] is not None:
                wb[(ch + 1) % 2].wait()
                wb[(ch + 1) % 2] = None
            pend = issue(ch + 1, *bufs[(ch + 1) % 2])
        for t in range(DCH):

            def body(j):
                sl = pl.ds(j * 16, 16)
                gz[t, sl] = gz[t, sl] + g0[t, sl] + g1[t, sl]

            plsc.parallel_loop(0, D // 16, 1, unroll=8)(body)
        wb[ch % 2] = pltpu.async_copy(
            gz, y_hbm.at[pl.ds(base + ch * DCH, DCH)], wsems[ch % 2])
    for w in wb:
        if w is not None:
            w.wait()


def _combine(outs, z, p0, p1):
    f32 = jnp.float32
    mesh = plsc.VectorSubcoreMesh(core_axis_name="c", subcore_axis_name="s")
    return pl.kernel(
        _combine_body,
        out_type=jax.ShapeDtypeStruct((T, D), f32),
        mesh=mesh,
        scratch_types=[
            pltpu.VMEM((DCH, D), f32),
            pltpu.VMEM((DCH, D), f32),
            pltpu.VMEM((DCH, D), f32),
            pltpu.VMEM((DCH, D), f32),
            pltpu.VMEM((DCH, D), f32),
            pltpu.VMEM((DCH, D), f32),
            pltpu.VMEM((DCH,), jnp.int32),
            pltpu.VMEM((DCH,), jnp.int32),
            pltpu.VMEM((DCH,), jnp.int32),
            pltpu.VMEM((DCH,), jnp.int32),
            pltpu.SemaphoreType.DMA,
            pltpu.SemaphoreType.DMA,
            pltpu.SemaphoreType.DMA,
            pltpu.SemaphoreType.DMA,
        ],
    )(outs, z, p0, p1)


def kernel(x, gate_w, ew1, eb1, ew2, eb2, ew3, eb3, sw1, sb1, sw2, sb2,
           sw3, sb3):
    a1, a2, w1t, w2t, rank, cnt = _route(x, gate_w)
    p0f, p1f, bem, bvm = _dispatch_meta(cnt, a1, a2, rank)
    p0 = p0f[:, 0]
    p1 = p1f[:, 0]
    bearr = bem[:NBR, 0]
    bvarr = bvm[:NBR, 0]
    xs, ws = _scatter_sorted(x, p0, p1, w1t, w2t)
    z = _shared_ffn(x, sw1, sb1.reshape(1, F), sw3, sb3.reshape(1, F),
                    sw2, sb2.reshape(1, D))
    outs = _group_ffn(bearr, bvarr, xs, ws, ew1, eb1, ew3, eb3, ew2, eb2)
    y = _combine(outs, z, p0, p1)
    return y


# speedup vs baseline: 1.6551x; 1.0313x over previous
import functools

import jax
import jax.numpy as jnp
from jax import lax
from jax.experimental import pallas as pl
from jax.experimental.pallas import tpu as pltpu
from jax.experimental.pallas import tpu_sc as plsc

T, D, F, E, K = 2048, 1024, 2048, 8, 2
LANES = 128
TB = 256
NTB = T // TB
BT = 256
NBR = -(-(T * K + E * (BT - 1)) // BT)
GR = NBR * BT
NC, NS = 2, 16
NW = NC * NS
TPW = T // NW
DCH = 16

_HI = lax.Precision.HIGHEST


def _fiota(shape, dim):
    return lax.broadcasted_iota(jnp.int32, shape, dim).astype(jnp.float32)


def _route_body(x_ref, gwt_ref,
                w1o_ref, w2o_ref, p0_ref, p1_ref, bem_ref, bvm_ref,
                carry, a1s, a2s, rks):
    pid = pl.program_id(0)

    @pl.when(pid == 0)
    def _():
        carry[...] = jnp.zeros_like(carry)

    @pl.when(pid < NTB)
    def _():
        x = x_ref[...]
        s = lax.dot_general(x, gwt_ref[...], (((1,), (1,)), ((), ())),
                            preferred_element_type=jnp.float32)
        lane = _fiota((TB, E), 1)
        mx = jnp.max(s, axis=1, keepdims=True)
        ex = jnp.exp(s - mx)
        p = ex / jnp.sum(ex, axis=1, keepdims=True)
        m1 = jnp.max(p, axis=1, keepdims=True)
        is1 = p == m1
        a1 = jnp.min(jnp.where(is1, lane, float(LANES)), axis=1, keepdims=True)
        not1 = lane != a1
        p2 = jnp.where(not1, p, -1.0)
        m2 = jnp.max(p2, axis=1, keepdims=True)
        is2 = jnp.logical_and(p2 == m2, not1)
        a2 = jnp.min(jnp.where(is2, lane, float(LANES)), axis=1, keepdims=True)

        ind = jnp.where(lane == a1, 1.0, 0.0) + jnp.where(lane == a2, 1.0, 0.0)
        row = _fiota((TB, TB), 0)
        col = _fiota((TB, TB), 1)
        ltri = jnp.where(row >= col, 1.0, 0.0)
        incl = lax.dot(ltri, ind)
        sl = pl.ds(pid * TB, TB)
        rks[sl, :] = incl - ind + carry[...]
        carry[...] = carry[...] + jnp.sum(ind, axis=0, keepdims=True)
        a1s[sl, :] = jnp.broadcast_to(a1, (TB, E))
        a2s[sl, :] = jnp.broadcast_to(a2, (TB, E))
        w1o_ref[...] = jnp.broadcast_to(m1, (TB, LANES))
        w2o_ref[...] = jnp.broadcast_to(m2, (TB, LANES))

    @pl.when(pid == NTB)
    def _():
        c = carry[...]
        rc = jnp.floor((c + float(BT - 1)) / float(BT)) * float(BT)
        ri = _fiota((E, E), 0)
        ci = _fiota((E, E), 1)
        ut = jnp.where(ri <= ci, 1.0, 0.0)
        cum_incl = lax.dot(rc, ut)
        cum_excl = cum_incl - rc

        lane = _fiota((T, E), 1)
        tot = cum_excl + rks[...]
        pos0 = jnp.sum(jnp.where(lane == a1s[...], tot, 0.0),
                       axis=1, keepdims=True)
        pos1 = jnp.sum(jnp.where(lane == a2s[...], tot, 0.0),
                       axis=1, keepdims=True)
        p0_ref[...] = jnp.broadcast_to(pos0, (T, E)).astype(jnp.int32)
        p1_ref[...] = jnp.broadcast_to(pos1, (T, E)).astype(jnp.int32)

        bj = _fiota((LANES, E), 0)
        start = bj * float(BT)
        endb = jnp.broadcast_to(cum_incl, (LANES, E))
        be = jnp.sum(jnp.where(start >= endb, 1.0, 0.0),
                     axis=1, keepdims=True)
        be = jnp.minimum(be, float(E - 1))
        bem_ref[...] = jnp.broadcast_to(be, (LANES, E)).astype(jnp.int32)
        exb = jnp.broadcast_to(cum_excl, (LANES, E))
        cb = jnp.broadcast_to(c, (LANES, E))
        has = jnp.logical_and(start >= exb, start < exb + cb)
        bv = jnp.sum(jnp.where(has, 1.0, 0.0), axis=1, keepdims=True)
        bvm_ref[...] = jnp.broadcast_to(bv, (LANES, E)).astype(jnp.int32)


def _route(x, gwt):
    f32 = jnp.float32
    i32 = jnp.int32
    blkc = lambda i: (jnp.minimum(i, NTB - 1), 0)
    whole = lambda i: (0, 0)
    return pl.pallas_call(
        _route_body,
        grid=(NTB + 1,),
        in_specs=[
            pl.BlockSpec((TB, D), blkc),
            pl.BlockSpec((E, D), whole),
        ],
        out_specs=(
            pl.BlockSpec((TB, LANES), blkc),
            pl.BlockSpec((TB, LANES), blkc),
            pl.BlockSpec((T, E), whole),
            pl.BlockSpec((T, E), whole),
            pl.BlockSpec((LANES, E), whole),
            pl.BlockSpec((LANES, E), whole),
        ),
        out_shape=(
            jax.ShapeDtypeStruct((T, LANES), f32),
            jax.ShapeDtypeStruct((T, LANES), f32),
            jax.ShapeDtypeStruct((T, E), i32),
            jax.ShapeDtypeStruct((T, E), i32),
            jax.ShapeDtypeStruct((LANES, E), i32),
            jax.ShapeDtypeStruct((LANES, E), i32),
        ),
        scratch_shapes=[pltpu.VMEM((1, E), f32), pltpu.VMEM((T, E), f32),
                        pltpu.VMEM((T, E), f32), pltpu.VMEM((T, E), f32)],
        compiler_params=pltpu.CompilerParams(
            dimension_semantics=("arbitrary",)),
    )(x, gwt)


def _shared_body(x_ref, sw1_ref, sb1_ref, sw3_ref, sb3_ref, sw2_ref, sb2_ref,
                 z_ref):
    x = x_ref[...]
    h1 = jnp.dot(x, sw1_ref[...], preferred_element_type=jnp.float32)
    h1 = h1 + sb1_ref[...]
    h3 = jnp.dot(x, sw3_ref[...], preferred_element_type=jnp.float32)
    h3 = h3 + sb3_ref[...]
    hh = (h1 * jax.nn.sigmoid(h1)) * h3
    z = jnp.dot(hh, sw2_ref[...], preferred_element_type=jnp.float32)
    z_ref[...] = z + sb2_ref[...]


def _shared_ffn(x, sw1, sb1, sw3, sb3, sw2, sb2):
    blk = lambda i: (i, 0)
    whole = lambda i: (0, 0)
    return pl.pallas_call(
        _shared_body,
        grid=(NTB,),
        in_specs=[
            pl.BlockSpec((TB, D), blk),
            pl.BlockSpec((D, F), whole),
            pl.BlockSpec((1, F), whole),
            pl.BlockSpec((D, F), whole),
            pl.BlockSpec((1, F), whole),
            pl.BlockSpec((F, D), whole),
            pl.BlockSpec((1, D), whole),
        ],
        out_specs=pl.BlockSpec((TB, D), blk),
        out_shape=jax.ShapeDtypeStruct((T, D), jnp.float32),
        compiler_params=pltpu.CompilerParams(
            dimension_semantics=("arbitrary",)),
    )(x, sw1, sb1, sw3, sb3, sw2, sb2)


def _scatter_body(x_hbm, p0_hbm, p1_hbm, w0_hbm, w1_hbm, xs_hbm, ws_hbm,
                  xv, i0, i1, w0v, w1v, sem):
    wid = lax.axis_index("s") * NC + lax.axis_index("c")
    base = wid * TPW
    pltpu.sync_copy(x_hbm.at[pl.ds(base, TPW)], xv)
    pltpu.sync_copy(p0_hbm.at[pl.ds(base, TPW)], i0)
    pltpu.sync_copy(p1_hbm.at[pl.ds(base, TPW)], i1)
    pltpu.sync_copy(w0_hbm.at[pl.ds(base, TPW)], w0v)
    pltpu.sync_copy(w1_hbm.at[pl.ds(base, TPW)], w1v)
    a = pltpu.async_copy(xv, xs_hbm.at[i0], sem)
    b = pltpu.async_copy(xv, xs_hbm.at[i1], sem)
    c = pltpu.async_copy(w0v, ws_hbm.at[i0], sem)
    d = pltpu.async_copy(w1v, ws_hbm.at[i1], sem)
    a.wait()
    b.wait()
    c.wait()
    d.wait()


def _scatter_sorted(x, p0, p1, w0r, w1r):
    mesh = plsc.VectorSubcoreMesh(core_axis_name="c", subcore_axis_name="s")
    return pl.kernel(
        _scatter_body,
        out_type=(jax.ShapeDtypeStruct((GR, D), jnp.float32),
                  jax.ShapeDtypeStruct((GR, LANES), jnp.float32)),
        mesh=mesh,
        scratch_types=[
            pltpu.VMEM((TPW, D), jnp.float32),
            pltpu.VMEM((TPW,), jnp.int32),
            pltpu.VMEM((TPW,), jnp.int32),
            pltpu.VMEM((TPW, LANES), jnp.float32),
            pltpu.VMEM((TPW, LANES), jnp.float32),
            pltpu.SemaphoreType.DMA,
        ],
    )(x, p0, p1, w0r, w1r)


def _group_ffn_body(be_ref, bv_ref, xs_hbm, ws_hbm, w1_hbm, b1_hbm, w3_hbm,
                    b3_hbm, w2_hbm, b2_hbm, o_hbm):
    def inner(xs_ref, ws_ref, w1_ref, b1_ref, w3_ref, b3_ref, w2_ref, b2_ref,
              o_ref):
        i = pl.program_id(0)

        @pl.when(bv_ref[i] > 0)
        def _():
            x = xs_ref[...]
            h1 = jnp.dot(x, w1_ref[0], preferred_element_type=jnp.float32)
            h1 = h1 + b1_ref[0]
            h3 = jnp.dot(x, w3_ref[0], preferred_element_type=jnp.float32)
            h3 = h3 + b3_ref[0]
            hh = (h1 * jax.nn.sigmoid(h1)) * h3
            o = jnp.dot(hh, w2_ref[0], preferred_element_type=jnp.float32)
            o_ref[...] = (o + b2_ref[0]) * ws_ref[:, :1]

    la = lambda: pl.Buffered(2, use_lookahead=True)
    pltpu.emit_pipeline(
        inner,
        grid=(NBR,),
        in_specs=[
            pl.BlockSpec((BT, D), lambda i: (i, 0)),
            pl.BlockSpec((BT, LANES), lambda i: (i, 0)),
            pl.BlockSpec((1, D, F), lambda i: (be_ref[i], 0, 0),
                         pipeline_mode=la()),
            pl.BlockSpec((1, 1, F), lambda i: (be_ref[i], 0, 0),
                         pipeline_mode=la()),
            pl.BlockSpec((1, D, F), lambda i: (be_ref[i], 0, 0),
                         pipeline_mode=la()),
            pl.BlockSpec((1, 1, F), lambda i: (be_ref[i], 0, 0),
                         pipeline_mode=la()),
            pl.BlockSpec((1, F, D), lambda i: (be_ref[i], 0, 0),
                         pipeline_mode=la()),
            pl.BlockSpec((1, 1, D), lambda i: (be_ref[i], 0, 0),
                         pipeline_mode=la()),
        ],
        out_specs=[pl.BlockSpec((BT, D), lambda i: (i, 0))],
    )(xs_hbm, ws_hbm, w1_hbm, b1_hbm, w3_hbm, b3_hbm, w2_hbm, b2_hbm, o_hbm)


def _group_ffn(bearr, bvarr, xs, ws, ew1, eb1, ew3, eb3, ew2, eb2):
    grid_spec = pltpu.PrefetchScalarGridSpec(
        num_scalar_prefetch=2,
        grid=(1,),
        in_specs=[pl.BlockSpec(memory_space=pl.ANY)] * 8,
        out_specs=pl.BlockSpec(memory_space=pl.ANY),
    )
    return pl.pallas_call(
        _group_ffn_body,
        grid_spec=grid_spec,
        out_shape=jax.ShapeDtypeStruct((GR, D), jnp.float32),
        compiler_params=pltpu.CompilerParams(
            dimension_semantics=("arbitrary",)),
    )(bearr, bvarr, xs, ws, ew1, eb1.reshape(E, 1, F), ew3,
      eb3.reshape(E, 1, F), ew2, eb2.reshape(E, 1, D))


def _combine_body(os_hbm, z_hbm, p0_hbm, p1_hbm, y_hbm,
                  g0a, g1a, gza, g0b, g1b, gzb, i0a, i1a, i0b, i1b,
                  sema, semb, wsa, wsb):
    wid = lax.axis_index("s") * NC + lax.axis_index("c")
    base = wid * TPW
    bufs = ((g0a, g1a, gza, i0a, i1a, sema),
            (g0b, g1b, gzb, i0b, i1b, semb))

    def issue(ch, g0, g1, gz, i0, i1, sem):
        tb = base + ch * DCH
        pltpu.sync_copy(p0_hbm.at[pl.ds(tb, DCH)], i0)
        pltpu.sync_copy(p1_hbm.at[pl.ds(tb, DCH)], i1)
        return (pltpu.async_copy(os_hbm.at[i0], g0, sem),
                pltpu.async_copy(os_hbm.at[i1], g1, sem),
                pltpu.async_copy(z_hbm.at[pl.ds(tb, DCH)], gz, sem))

    nch = TPW // DCH
    wsems = (wsa, wsb)
    pend = issue(0, *bufs[0])
    wb = [None, None]
    for ch in range(nch):
        g0, g1, gz = bufs[ch % 2][:3]
        for cp in pend:
            cp.wait()
        if ch + 1 < nch:
            if wb[(ch + 1) % 2] is not None:
                wb[(ch + 1) % 2].wait()
                wb[(ch + 1) % 2] = None
            pend = issue(ch + 1, *bufs[(ch + 1) % 2])
        for t in range(DCH):

            def body(j):
                sl = pl.ds(j * 16, 16)
                gz[t, sl] = gz[t, sl] + g0[t, sl] + g1[t, sl]

            plsc.parallel_loop(0, D // 16, 1, unroll=8)(body)
        wb[ch % 2] = pltpu.async_copy(
            gz, y_hbm.at[pl.ds(base + ch * DCH, DCH)], wsems[ch % 2])
    for w in wb:
        if w is not None:
            w.wait()


def _combine(outs, z, p0, p1):
    f32 = jnp.float32
    mesh = plsc.VectorSubcoreMesh(core_axis_name="c", subcore_axis_name="s")
    return pl.kernel(
        _combine_body,
        out_type=jax.ShapeDtypeStruct((T, D), f32),
        mesh=mesh,
        scratch_types=[
            pltpu.VMEM((DCH, D), f32),
            pltpu.VMEM((DCH, D), f32),
            pltpu.VMEM((DCH, D), f32),
            pltpu.VMEM((DCH, D), f32),
            pltpu.VMEM((DCH, D), f32),
            pltpu.VMEM((DCH, D), f32),
            pltpu.VMEM((DCH,), jnp.int32),
            pltpu.VMEM((DCH,), jnp.int32),
            pltpu.VMEM((DCH,), jnp.int32),
            pltpu.VMEM((DCH,), jnp.int32),
            pltpu.SemaphoreType.DMA,
            pltpu.SemaphoreType.DMA,
            pltpu.SemaphoreType.DMA,
            pltpu.SemaphoreType.DMA,
        ],
    )(outs, z, p0, p1)


def kernel(x, gate_w, ew1, eb1, ew2, eb2, ew3, eb3, sw1, sb1, sw2, sb2,
           sw3, sb3):
    w1t, w2t, p0f, p1f, bem, bvm = _route(x, gate_w)
    p0 = p0f[:, 0]
    p1 = p1f[:, 0]
    bearr = bem[:NBR, 0]
    bvarr = bvm[:NBR, 0]
    xs, ws = _scatter_sorted(x, p0, p1, w1t, w2t)
    z = _shared_ffn(x, sw1, sb1.reshape(1, F), sw3, sb3.reshape(1, F),
                    sw2, sb2.reshape(1, D))
    outs = _group_ffn(bearr, bvarr, xs, ws, ew1, eb1, ew3, eb3, ew2, eb2)
    y = _combine(outs, z, p0, p1)
    return y


# speedup vs baseline: 1.6814x; 1.0159x over previous
import functools

import jax
import jax.numpy as jnp
from jax import lax
from jax.experimental import pallas as pl
from jax.experimental.pallas import tpu as pltpu
from jax.experimental.pallas import tpu_sc as plsc

T, D, F, E, K = 2048, 1024, 2048, 8, 2
LANES = 128
TB = 256
NTB = T // TB
BT = 256
NBR = -(-(T * K + E * (BT - 1)) // BT)
GR = NBR * BT
NC, NS = 2, 16
NW = NC * NS
TPW = T // NW
DCH = 16

_HI = lax.Precision.HIGHEST


def _fiota(shape, dim):
    return lax.broadcasted_iota(jnp.int32, shape, dim).astype(jnp.float32)


def _route_body(x_ref, gwt_ref,
                w1o_ref, w2o_ref, p0_ref, p1_ref, bem_ref, bvm_ref,
                carry, a1s, a2s, rks):
    pid = pl.program_id(0)

    @pl.when(pid == 0)
    def _():
        carry[...] = jnp.zeros_like(carry)

    @pl.when(pid < NTB)
    def _():
        x = x_ref[...]
        s = lax.dot_general(x, gwt_ref[...], (((1,), (1,)), ((), ())),
                            preferred_element_type=jnp.float32)
        lane = _fiota((TB, E), 1)
        mx = jnp.max(s, axis=1, keepdims=True)
        ex = jnp.exp(s - mx)
        p = ex / jnp.sum(ex, axis=1, keepdims=True)
        m1 = jnp.max(p, axis=1, keepdims=True)
        is1 = p == m1
        a1 = jnp.min(jnp.where(is1, lane, float(LANES)), axis=1, keepdims=True)
        not1 = lane != a1
        p2 = jnp.where(not1, p, -1.0)
        m2 = jnp.max(p2, axis=1, keepdims=True)
        is2 = jnp.logical_and(p2 == m2, not1)
        a2 = jnp.min(jnp.where(is2, lane, float(LANES)), axis=1, keepdims=True)

        ind = jnp.where(lane == a1, 1.0, 0.0) + jnp.where(lane == a2, 1.0, 0.0)
        row = _fiota((TB, TB), 0)
        col = _fiota((TB, TB), 1)
        ltri = jnp.where(row >= col, 1.0, 0.0)
        incl = lax.dot(ltri, ind)
        sl = pl.ds(pid * TB, TB)
        rks[sl, :] = incl - ind + carry[...]
        carry[...] = carry[...] + jnp.sum(ind, axis=0, keepdims=True)
        a1s[sl, :] = jnp.broadcast_to(a1, (TB, E))
        a2s[sl, :] = jnp.broadcast_to(a2, (TB, E))
        w1o_ref[...] = jnp.broadcast_to(m1, (TB, LANES))
        w2o_ref[...] = jnp.broadcast_to(m2, (TB, LANES))

    @pl.when(pid == NTB)
    def _():
        c = carry[...]
        rc = jnp.floor((c + float(BT - 1)) / float(BT)) * float(BT)
        ri = _fiota((E, E), 0)
        ci = _fiota((E, E), 1)
        ut = jnp.where(ri <= ci, 1.0, 0.0)
        cum_incl = lax.dot(rc, ut)
        cum_excl = cum_incl - rc

        lane = _fiota((T, E), 1)
        tot = cum_excl + rks[...]
        pos0 = jnp.sum(jnp.where(lane == a1s[...], tot, 0.0),
                       axis=1, keepdims=True)
        pos1 = jnp.sum(jnp.where(lane == a2s[...], tot, 0.0),
                       axis=1, keepdims=True)
        p0_ref[...] = jnp.broadcast_to(pos0, (T, E)).astype(jnp.int32)
        p1_ref[...] = jnp.broadcast_to(pos1, (T, E)).astype(jnp.int32)

        bj = _fiota((LANES, E), 0)
        start = bj * float(BT)
        endb = jnp.broadcast_to(cum_incl, (LANES, E))
        be = jnp.sum(jnp.where(start >= endb, 1.0, 0.0),
                     axis=1, keepdims=True)
        be = jnp.minimum(be, float(E - 1))
        bem_ref[...] = jnp.broadcast_to(be, (LANES, E)).astype(jnp.int32)
        exb = jnp.broadcast_to(cum_excl, (LANES, E))
        cb = jnp.broadcast_to(c, (LANES, E))
        has = jnp.logical_and(start >= exb, start < exb + cb)
        bv = jnp.sum(jnp.where(has, 1.0, 0.0), axis=1, keepdims=True)
        bvm_ref[...] = jnp.broadcast_to(bv, (LANES, E)).astype(jnp.int32)


def _route(x, gwt):
    f32 = jnp.float32
    i32 = jnp.int32
    blkc = lambda i: (jnp.minimum(i, NTB - 1), 0)
    whole = lambda i: (0, 0)
    return pl.pallas_call(
        _route_body,
        grid=(NTB + 1,),
        in_specs=[
            pl.BlockSpec((TB, D), blkc),
            pl.BlockSpec((E, D), whole),
        ],
        out_specs=(
            pl.BlockSpec((TB, LANES), blkc),
            pl.BlockSpec((TB, LANES), blkc),
            pl.BlockSpec((T, E), whole),
            pl.BlockSpec((T, E), whole),
            pl.BlockSpec((LANES, E), whole),
            pl.BlockSpec((LANES, E), whole),
        ),
        out_shape=(
            jax.ShapeDtypeStruct((T, LANES), f32),
            jax.ShapeDtypeStruct((T, LANES), f32),
            jax.ShapeDtypeStruct((T, E), i32),
            jax.ShapeDtypeStruct((T, E), i32),
            jax.ShapeDtypeStruct((LANES, E), i32),
            jax.ShapeDtypeStruct((LANES, E), i32),
        ),
        scratch_shapes=[pltpu.VMEM((1, E), f32), pltpu.VMEM((T, E), f32),
                        pltpu.VMEM((T, E), f32), pltpu.VMEM((T, E), f32)],
        compiler_params=pltpu.CompilerParams(
            dimension_semantics=("arbitrary",)),
    )(x, gwt)


def _shared_body(x_ref, sw1_ref, sb1_ref, sw3_ref, sb3_ref, sw2_ref, sb2_ref,
                 z_ref):
    x = x_ref[...]
    h1 = jnp.dot(x, sw1_ref[...], preferred_element_type=jnp.float32)
    h1 = h1 + sb1_ref[...]
    h3 = jnp.dot(x, sw3_ref[...], preferred_element_type=jnp.float32)
    h3 = h3 + sb3_ref[...]
    hh = (h1 * jax.nn.sigmoid(h1)) * h3
    z = jnp.dot(hh, sw2_ref[...], preferred_element_type=jnp.float32)
    z_ref[...] = z + sb2_ref[...]


def _shared_ffn(x, sw1, sb1, sw3, sb3, sw2, sb2):
    blk = lambda i: (i, 0)
    whole = lambda i: (0, 0)
    return pl.pallas_call(
        _shared_body,
        grid=(NTB,),
        in_specs=[
            pl.BlockSpec((TB, D), blk),
            pl.BlockSpec((D, F), whole),
            pl.BlockSpec((1, F), whole),
            pl.BlockSpec((D, F), whole),
            pl.BlockSpec((1, F), whole),
            pl.BlockSpec((F, D), whole),
            pl.BlockSpec((1, D), whole),
        ],
        out_specs=pl.BlockSpec((TB, D), blk),
        out_shape=jax.ShapeDtypeStruct((T, D), jnp.float32),
        compiler_params=pltpu.CompilerParams(
            dimension_semantics=("arbitrary",)),
    )(x, sw1, sb1, sw3, sb3, sw2, sb2)


def _scatter_body(x_hbm, p0_hbm, p1_hbm, w0_hbm, w1_hbm, xs_hbm, ws_hbm,
                  xv, i0, i1, w0v, w1v, sem):
    wid = lax.axis_index("s") * NC + lax.axis_index("c")
    base = wid * TPW
    pltpu.sync_copy(x_hbm.at[pl.ds(base, TPW)], xv)
    pltpu.sync_copy(p0_hbm.at[pl.ds(base, TPW)], i0)
    pltpu.sync_copy(p1_hbm.at[pl.ds(base, TPW)], i1)
    pltpu.sync_copy(w0_hbm.at[pl.ds(base, TPW)], w0v)
    pltpu.sync_copy(w1_hbm.at[pl.ds(base, TPW)], w1v)
    a = pltpu.async_copy(xv, xs_hbm.at[i0], sem)
    b = pltpu.async_copy(xv, xs_hbm.at[i1], sem)
    c = pltpu.async_copy(w0v, ws_hbm.at[i0], sem)
    d = pltpu.async_copy(w1v, ws_hbm.at[i1], sem)
    a.wait()
    b.wait()
    c.wait()
    d.wait()


def _scatter_sorted(x, p0, p1, w0r, w1r):
    mesh = plsc.VectorSubcoreMesh(core_axis_name="c", subcore_axis_name="s")
    return pl.kernel(
        _scatter_body,
        out_type=(jax.ShapeDtypeStruct((GR, D), jnp.float32),
                  jax.ShapeDtypeStruct((GR, LANES), jnp.float32)),
        mesh=mesh,
        scratch_types=[
            pltpu.VMEM((TPW, D), jnp.float32),
            pltpu.VMEM((TPW,), jnp.int32),
            pltpu.VMEM((TPW,), jnp.int32),
            pltpu.VMEM((TPW, LANES), jnp.float32),
            pltpu.VMEM((TPW, LANES), jnp.float32),
            pltpu.SemaphoreType.DMA,
        ],
    )(x, p0, p1, w0r, w1r)


def _group_ffn_body(be_ref, bv_ref, xs_hbm, ws_hbm, w1_hbm, b1_hbm, w3_hbm,
                    b3_hbm, w2_hbm, b2_hbm, o_hbm):
    def inner(xs_ref, ws_ref, w1_ref, b1_ref, w3_ref, b3_ref, w2_ref, b2_ref,
              o_ref):
        i = pl.program_id(0)

        @pl.when(bv_ref[i] > 0)
        def _():
            x = xs_ref[...]
            h1 = jnp.dot(x, w1_ref[0], preferred_element_type=jnp.float32)
            h1 = h1 + b1_ref[...]
            h3 = jnp.dot(x, w3_ref[0], preferred_element_type=jnp.float32)
            h3 = h3 + b3_ref[...]
            hh = (h1 * jax.nn.sigmoid(h1)) * h3
            o = jnp.dot(hh, w2_ref[0], preferred_element_type=jnp.float32)
            o_ref[...] = (o + b2_ref[...]) * ws_ref[:, :1]

    la = lambda: pl.Buffered(2, use_lookahead=True)
    pltpu.emit_pipeline(
        inner,
        grid=(NBR,),
        in_specs=[
            pl.BlockSpec((BT, D), lambda i: (i, 0)),
            pl.BlockSpec((BT, LANES), lambda i: (i, 0)),
            pl.BlockSpec((1, D, F), lambda i: (be_ref[i], 0, 0),
                         pipeline_mode=la()),
            pl.BlockSpec((1, F), lambda i: (be_ref[i], 0),
                         pipeline_mode=la()),
            pl.BlockSpec((1, D, F), lambda i: (be_ref[i], 0, 0),
                         pipeline_mode=la()),
            pl.BlockSpec((1, F), lambda i: (be_ref[i], 0),
                         pipeline_mode=la()),
            pl.BlockSpec((1, F, D), lambda i: (be_ref[i], 0, 0),
                         pipeline_mode=la()),
            pl.BlockSpec((1, D), lambda i: (be_ref[i], 0),
                         pipeline_mode=la()),
        ],
        out_specs=[pl.BlockSpec((BT, D), lambda i: (i, 0))],
    )(xs_hbm, ws_hbm, w1_hbm, b1_hbm, w3_hbm, b3_hbm, w2_hbm, b2_hbm, o_hbm)


def _group_ffn(bearr, bvarr, xs, ws, ew1, eb1, ew3, eb3, ew2, eb2):
    grid_spec = pltpu.PrefetchScalarGridSpec(
        num_scalar_prefetch=2,
        grid=(1,),
        in_specs=[pl.BlockSpec(memory_space=pl.ANY)] * 8,
        out_specs=pl.BlockSpec(memory_space=pl.ANY),
    )
    return pl.pallas_call(
        _group_ffn_body,
        grid_spec=grid_spec,
        out_shape=jax.ShapeDtypeStruct((GR, D), jnp.float32),
        compiler_params=pltpu.CompilerParams(
            dimension_semantics=("arbitrary",)),
    )(bearr, bvarr, xs, ws, ew1, eb1, ew3, eb3, ew2, eb2)


def _combine_body(os_hbm, z_hbm, p0_hbm, p1_hbm, y_hbm,
                  g0a, g1a, gza, g0b, g1b, gzb, i0a, i1a, i0b, i1b,
                  sema, semb, wsa, wsb):
    wid = lax.axis_index("s") * NC + lax.axis_index("c")
    base = wid * TPW
    bufs = ((g0a, g1a, gza, i0a, i1a, sema),
            (g0b, g1b, gzb, i0b, i1b, semb))

    def issue(ch, g0, g1, gz, i0, i1, sem):
        tb = base + ch * DCH
        pltpu.sync_copy(p0_hbm.at[pl.ds(tb, DCH)], i0)
        pltpu.sync_copy(p1_hbm.at[pl.ds(tb, DCH)], i1)
        return (pltpu.async_copy(os_hbm.at[i0], g0, sem),
                pltpu.async_copy(os_hbm.at[i1], g1, sem),
                pltpu.async_copy(z_hbm.at[pl.ds(tb, DCH)], gz, sem))

    nch = TPW // DCH
    wsems = (wsa, wsb)
    pend = issue(0, *bufs[0])
    wb = [None, None]
    for ch in range(nch):
        g0, g1, gz = bufs[ch % 2][:3]
        for cp in pend:
            cp.wait()
        if ch + 1 < nch:
            if wb[(ch + 1) % 2] is not None:
                wb[(ch + 1) % 2].wait()
                wb[(ch + 1) % 2] = None
            pend = issue(ch + 1, *bufs[(ch + 1) % 2])
        for t in range(DCH):

            def body(j):
                sl = pl.ds(j * 16, 16)
                gz[t, sl] = gz[t, sl] + g0[t, sl] + g1[t, sl]

            plsc.parallel_loop(0, D // 16, 1, unroll=8)(body)
        wb[ch % 2] = pltpu.async_copy(
            gz, y_hbm.at[pl.ds(base + ch * DCH, DCH)], wsems[ch % 2])
    for w in wb:
        if w is not None:
            w.wait()


def _combine(outs, z, p0, p1):
    f32 = jnp.float32
    mesh = plsc.VectorSubcoreMesh(core_axis_name="c", subcore_axis_name="s")
    return pl.kernel(
        _combine_body,
        out_type=jax.ShapeDtypeStruct((T, D), f32),
        mesh=mesh,
        scratch_types=[
            pltpu.VMEM((DCH, D), f32),
            pltpu.VMEM((DCH, D), f32),
            pltpu.VMEM((DCH, D), f32),
            pltpu.VMEM((DCH, D), f32),
            pltpu.VMEM((DCH, D), f32),
            pltpu.VMEM((DCH, D), f32),
            pltpu.VMEM((DCH,), jnp.int32),
            pltpu.VMEM((DCH,), jnp.int32),
            pltpu.VMEM((DCH,), jnp.int32),
            pltpu.VMEM((DCH,), jnp.int32),
            pltpu.SemaphoreType.DMA,
            pltpu.SemaphoreType.DMA,
            pltpu.SemaphoreType.DMA,
            pltpu.SemaphoreType.DMA,
        ],
    )(outs, z, p0, p1)


def kernel(x, gate_w, ew1, eb1, ew2, eb2, ew3, eb3, sw1, sb1, sw2, sb2,
           sw3, sb3):
    w1t, w2t, p0f, p1f, bem, bvm = _route(x, gate_w)
    p0 = p0f[:, 0]
    p1 = p1f[:, 0]
    bearr = bem[:NBR, 0]
    bvarr = bvm[:NBR, 0]
    xs, ws = _scatter_sorted(x, p0, p1, w1t, w2t)
    z = _shared_ffn(x, sw1, sb1.reshape(1, F), sw3, sb3.reshape(1, F),
                    sw2, sb2.reshape(1, D))
    outs = _group_ffn(bearr, bvarr, xs, ws, ew1, eb1, ew3, eb3, ew2, eb2)
    y = _combine(outs, z, p0, p1)
    return y


# speedup vs baseline: 1.6831x; 1.0010x over previous
import jax
import jax.numpy as jnp
from jax import lax
from jax.experimental import pallas as pl
from jax.experimental.pallas import tpu as pltpu
from jax.experimental.pallas import tpu_sc as plsc

T, D, F, E, K = 2048, 1024, 2048, 8, 2
LANES = 128
TB = 256
NTB = T // TB
BT = 256
NBR = -(-(T * K + E * (BT - 1)) // BT)
GR = NBR * BT
NC, NS = 2, 16
NW = NC * NS
TPW = T // NW
DCH = 16

def _fiota(shape, dim):
    return lax.broadcasted_iota(jnp.int32, shape, dim).astype(jnp.float32)


def _route_body(x_ref, gwt_ref,
                w1o_ref, w2o_ref, p0_ref, p1_ref, bem_ref, bvm_ref,
                carry, a1s, a2s, rks):
    pid = pl.program_id(0)

    @pl.when(pid == 0)
    def _():
        carry[...] = jnp.zeros_like(carry)

    @pl.when(pid < NTB)
    def _():
        x = x_ref[...]
        s = lax.dot_general(x, gwt_ref[...], (((1,), (1,)), ((), ())),
                            preferred_element_type=jnp.float32)
        lane = _fiota((TB, E), 1)
        mx = jnp.max(s, axis=1, keepdims=True)
        ex = jnp.exp(s - mx)
        p = ex / jnp.sum(ex, axis=1, keepdims=True)
        m1 = jnp.max(p, axis=1, keepdims=True)
        is1 = p == m1
        a1 = jnp.min(jnp.where(is1, lane, float(LANES)), axis=1, keepdims=True)
        not1 = lane != a1
        p2 = jnp.where(not1, p, -1.0)
        m2 = jnp.max(p2, axis=1, keepdims=True)
        is2 = jnp.logical_and(p2 == m2, not1)
        a2 = jnp.min(jnp.where(is2, lane, float(LANES)), axis=1, keepdims=True)

        ind = jnp.where(lane == a1, 1.0, 0.0) + jnp.where(lane == a2, 1.0, 0.0)
        row = _fiota((TB, TB), 0)
        col = _fiota((TB, TB), 1)
        ltri = jnp.where(row >= col, 1.0, 0.0)
        incl = lax.dot(ltri, ind)
        sl = pl.ds(pid * TB, TB)
        rks[sl, :] = incl - ind + carry[...]
        carry[...] = carry[...] + jnp.sum(ind, axis=0, keepdims=True)
        a1s[sl, :] = jnp.broadcast_to(a1, (TB, E))
        a2s[sl, :] = jnp.broadcast_to(a2, (TB, E))
        w1o_ref[...] = jnp.broadcast_to(m1, (TB, LANES))
        w2o_ref[...] = jnp.broadcast_to(m2, (TB, LANES))

    @pl.when(pid == NTB)
    def _():
        c = carry[...]
        rc = jnp.floor((c + float(BT - 1)) / float(BT)) * float(BT)
        ri = _fiota((E, E), 0)
        ci = _fiota((E, E), 1)
        ut = jnp.where(ri <= ci, 1.0, 0.0)
        cum_incl = lax.dot(rc, ut)
        cum_excl = cum_incl - rc

        lane = _fiota((T, E), 1)
        tot = cum_excl + rks[...]
        pos0 = jnp.sum(jnp.where(lane == a1s[...], tot, 0.0),
                       axis=1, keepdims=True)
        pos1 = jnp.sum(jnp.where(lane == a2s[...], tot, 0.0),
                       axis=1, keepdims=True)
        p0_ref[...] = jnp.broadcast_to(pos0, (T, E)).astype(jnp.int32)
        p1_ref[...] = jnp.broadcast_to(pos1, (T, E)).astype(jnp.int32)

        bj = _fiota((LANES, E), 0)
        start = bj * float(BT)
        endb = jnp.broadcast_to(cum_incl, (LANES, E))
        be = jnp.sum(jnp.where(start >= endb, 1.0, 0.0),
                     axis=1, keepdims=True)
        be = jnp.minimum(be, float(E - 1))
        bem_ref[...] = jnp.broadcast_to(be, (LANES, E)).astype(jnp.int32)
        exb = jnp.broadcast_to(cum_excl, (LANES, E))
        cb = jnp.broadcast_to(c, (LANES, E))
        has = jnp.logical_and(start >= exb, start < exb + cb)
        bv = jnp.sum(jnp.where(has, 1.0, 0.0), axis=1, keepdims=True)
        bvm_ref[...] = jnp.broadcast_to(bv, (LANES, E)).astype(jnp.int32)


def _route(x, gwt):
    f32 = jnp.float32
    i32 = jnp.int32
    blkc = lambda i: (jnp.minimum(i, NTB - 1), 0)
    whole = lambda i: (0, 0)
    return pl.pallas_call(
        _route_body,
        grid=(NTB + 1,),
        in_specs=[
            pl.BlockSpec((TB, D), blkc),
            pl.BlockSpec((E, D), whole),
        ],
        out_specs=(
            pl.BlockSpec((TB, LANES), blkc),
            pl.BlockSpec((TB, LANES), blkc),
            pl.BlockSpec((T, E), whole),
            pl.BlockSpec((T, E), whole),
            pl.BlockSpec((LANES, E), whole),
            pl.BlockSpec((LANES, E), whole),
        ),
        out_shape=(
            jax.ShapeDtypeStruct((T, LANES), f32),
            jax.ShapeDtypeStruct((T, LANES), f32),
            jax.ShapeDtypeStruct((T, E), i32),
            jax.ShapeDtypeStruct((T, E), i32),
            jax.ShapeDtypeStruct((LANES, E), i32),
            jax.ShapeDtypeStruct((LANES, E), i32),
        ),
        scratch_shapes=[pltpu.VMEM((1, E), f32), pltpu.VMEM((T, E), f32),
                        pltpu.VMEM((T, E), f32), pltpu.VMEM((T, E), f32)],
        compiler_params=pltpu.CompilerParams(
            dimension_semantics=("arbitrary",)),
    )(x, gwt)


def _shared_body(x_ref, sw1_ref, sb1_ref, sw3_ref, sb3_ref, sw2_ref, sb2_ref,
                 z_ref):
    x = x_ref[...]
    h1 = jnp.dot(x, sw1_ref[...], preferred_element_type=jnp.float32)
    h1 = h1 + sb1_ref[...]
    h3 = jnp.dot(x, sw3_ref[...], preferred_element_type=jnp.float32)
    h3 = h3 + sb3_ref[...]
    hh = (h1 * jax.nn.sigmoid(h1)) * h3
    z = jnp.dot(hh, sw2_ref[...], preferred_element_type=jnp.float32)
    z_ref[...] = z + sb2_ref[...]


def _shared_ffn(x, sw1, sb1, sw3, sb3, sw2, sb2):
    blk = lambda i: (i, 0)
    whole = lambda i: (0, 0)
    return pl.pallas_call(
        _shared_body,
        grid=(NTB,),
        in_specs=[
            pl.BlockSpec((TB, D), blk),
            pl.BlockSpec((D, F), whole),
            pl.BlockSpec((1, F), whole),
            pl.BlockSpec((D, F), whole),
            pl.BlockSpec((1, F), whole),
            pl.BlockSpec((F, D), whole),
            pl.BlockSpec((1, D), whole),
        ],
        out_specs=pl.BlockSpec((TB, D), blk),
        out_shape=jax.ShapeDtypeStruct((T, D), jnp.float32),
        compiler_params=pltpu.CompilerParams(
            dimension_semantics=("arbitrary",)),
    )(x, sw1, sb1, sw3, sb3, sw2, sb2)


def _scatter_body(x_hbm, p0_hbm, p1_hbm, w0_hbm, w1_hbm, xs_hbm, ws_hbm,
                  xv, i0, i1, w0v, w1v, sem):
    wid = lax.axis_index("s") * NC + lax.axis_index("c")
    base = wid * TPW
    pltpu.sync_copy(x_hbm.at[pl.ds(base, TPW)], xv)
    pltpu.sync_copy(p0_hbm.at[pl.ds(base, TPW)], i0)
    pltpu.sync_copy(p1_hbm.at[pl.ds(base, TPW)], i1)
    pltpu.sync_copy(w0_hbm.at[pl.ds(base, TPW)], w0v)
    pltpu.sync_copy(w1_hbm.at[pl.ds(base, TPW)], w1v)
    a = pltpu.async_copy(xv, xs_hbm.at[i0], sem)
    b = pltpu.async_copy(xv, xs_hbm.at[i1], sem)
    c = pltpu.async_copy(w0v, ws_hbm.at[i0], sem)
    d = pltpu.async_copy(w1v, ws_hbm.at[i1], sem)
    a.wait()
    b.wait()
    c.wait()
    d.wait()


def _scatter_sorted(x, p0, p1, w0r, w1r):
    mesh = plsc.VectorSubcoreMesh(core_axis_name="c", subcore_axis_name="s")
    return pl.kernel(
        _scatter_body,
        out_type=(jax.ShapeDtypeStruct((GR, D), jnp.float32),
                  jax.ShapeDtypeStruct((GR, LANES), jnp.float32)),
        mesh=mesh,
        scratch_types=[
            pltpu.VMEM((TPW, D), jnp.float32),
            pltpu.VMEM((TPW,), jnp.int32),
            pltpu.VMEM((TPW,), jnp.int32),
            pltpu.VMEM((TPW, LANES), jnp.float32),
            pltpu.VMEM((TPW, LANES), jnp.float32),
            pltpu.SemaphoreType.DMA,
        ],
    )(x, p0, p1, w0r, w1r)


def _group_ffn_body(be_ref, bv_ref, xs_hbm, ws_hbm, w1_hbm, b1_hbm, w3_hbm,
                    b3_hbm, w2_hbm, b2_hbm, o_hbm):
    def inner(xs_ref, ws_ref, w1_ref, b1_ref, w3_ref, b3_ref, w2_ref, b2_ref,
              o_ref):
        i = pl.program_id(0)

        @pl.when(bv_ref[i] > 0)
        def _():
            x = xs_ref[...]
            h1 = jnp.dot(x, w1_ref[0], preferred_element_type=jnp.float32)
            h1 = h1 + b1_ref[...]
            h3 = jnp.dot(x, w3_ref[0], preferred_element_type=jnp.float32)
            h3 = h3 + b3_ref[...]
            hh = (h1 * jax.nn.sigmoid(h1)) * h3
            o = jnp.dot(hh, w2_ref[0], preferred_element_type=jnp.float32)
            o_ref[...] = (o + b2_ref[...]) * ws_ref[:, :1]

    la = lambda: pl.Buffered(2, use_lookahead=True)
    pltpu.emit_pipeline(
        inner,
        grid=(NBR,),
        in_specs=[
            pl.BlockSpec((BT, D), lambda i: (i, 0)),
            pl.BlockSpec((BT, LANES), lambda i: (i, 0)),
            pl.BlockSpec((1, D, F), lambda i: (be_ref[i], 0, 0),
                         pipeline_mode=la()),
            pl.BlockSpec((1, F), lambda i: (be_ref[i], 0),
                         pipeline_mode=la()),
            pl.BlockSpec((1, D, F), lambda i: (be_ref[i], 0, 0),
                         pipeline_mode=la()),
            pl.BlockSpec((1, F), lambda i: (be_ref[i], 0),
                         pipeline_mode=la()),
            pl.BlockSpec((1, F, D), lambda i: (be_ref[i], 0, 0),
                         pipeline_mode=la()),
            pl.BlockSpec((1, D), lambda i: (be_ref[i], 0),
                         pipeline_mode=la()),
        ],
        out_specs=[pl.BlockSpec((BT, D), lambda i: (i, 0))],
    )(xs_hbm, ws_hbm, w1_hbm, b1_hbm, w3_hbm, b3_hbm, w2_hbm, b2_hbm, o_hbm)


def _group_ffn(bearr, bvarr, xs, ws, ew1, eb1, ew3, eb3, ew2, eb2):
    grid_spec = pltpu.PrefetchScalarGridSpec(
        num_scalar_prefetch=2,
        grid=(1,),
        in_specs=[pl.BlockSpec(memory_space=pl.ANY)] * 8,
        out_specs=pl.BlockSpec(memory_space=pl.ANY),
    )
    return pl.pallas_call(
        _group_ffn_body,
        grid_spec=grid_spec,
        out_shape=jax.ShapeDtypeStruct((GR, D), jnp.float32),
        compiler_params=pltpu.CompilerParams(
            dimension_semantics=("arbitrary",)),
    )(bearr, bvarr, xs, ws, ew1, eb1, ew3, eb3, ew2, eb2)


def _combine_body(os_hbm, z_hbm, p0_hbm, p1_hbm, y_hbm,
                  g0a, g1a, gza, g0b, g1b, gzb, i0a, i1a, i0b, i1b,
                  sema, semb, wsa, wsb):
    wid = lax.axis_index("s") * NC + lax.axis_index("c")
    base = wid * TPW
    bufs = ((g0a, g1a, gza, i0a, i1a, sema),
            (g0b, g1b, gzb, i0b, i1b, semb))

    def issue(ch, g0, g1, gz, i0, i1, sem):
        tb = base + ch * DCH
        pltpu.sync_copy(p0_hbm.at[pl.ds(tb, DCH)], i0)
        pltpu.sync_copy(p1_hbm.at[pl.ds(tb, DCH)], i1)
        return (pltpu.async_copy(os_hbm.at[i0], g0, sem),
                pltpu.async_copy(os_hbm.at[i1], g1, sem),
                pltpu.async_copy(z_hbm.at[pl.ds(tb, DCH)], gz, sem))

    nch = TPW // DCH
    wsems = (wsa, wsb)
    pend = issue(0, *bufs[0])
    wb = [None, None]
    for ch in range(nch):
        g0, g1, gz = bufs[ch % 2][:3]
        for cp in pend:
            cp.wait()
        if ch + 1 < nch:
            if wb[(ch + 1) % 2] is not None:
                wb[(ch + 1) % 2].wait()
                wb[(ch + 1) % 2] = None
            pend = issue(ch + 1, *bufs[(ch + 1) % 2])
        for t in range(DCH):

            def body(j):
                sl = pl.ds(j * 16, 16)
                gz[t, sl] = gz[t, sl] + g0[t, sl] + g1[t, sl]

            plsc.parallel_loop(0, D // 16, 1, unroll=8)(body)
        wb[ch % 2] = pltpu.async_copy(
            gz, y_hbm.at[pl.ds(base + ch * DCH, DCH)], wsems[ch % 2])
    for w in wb:
        if w is not None:
            w.wait()


def _combine(outs, z, p0, p1):
    f32 = jnp.float32
    mesh = plsc.VectorSubcoreMesh(core_axis_name="c", subcore_axis_name="s")
    return pl.kernel(
        _combine_body,
        out_type=jax.ShapeDtypeStruct((T, D), f32),
        mesh=mesh,
        scratch_types=[
            pltpu.VMEM((DCH, D), f32),
            pltpu.VMEM((DCH, D), f32),
            pltpu.VMEM((DCH, D), f32),
            pltpu.VMEM((DCH, D), f32),
            pltpu.VMEM((DCH, D), f32),
            pltpu.VMEM((DCH, D), f32),
            pltpu.VMEM((DCH,), jnp.int32),
            pltpu.VMEM((DCH,), jnp.int32),
            pltpu.VMEM((DCH,), jnp.int32),
            pltpu.VMEM((DCH,), jnp.int32),
            pltpu.SemaphoreType.DMA,
            pltpu.SemaphoreType.DMA,
            pltpu.SemaphoreType.DMA,
            pltpu.SemaphoreType.DMA,
        ],
    )(outs, z, p0, p1)


def kernel(x, gate_w, ew1, eb1, ew2, eb2, ew3, eb3, sw1, sb1, sw2, sb2,
           sw3, sb3):
    w1t, w2t, p0f, p1f, bem, bvm = _route(x, gate_w)
    p0 = p0f[:, 0]
    p1 = p1f[:, 0]
    bearr = bem[:NBR, 0]
    bvarr = bvm[:NBR, 0]
    xs, ws = _scatter_sorted(x, p0, p1, w1t, w2t)
    z = _shared_ffn(x, sw1, sb1.reshape(1, F), sw3, sb3.reshape(1, F),
                    sw2, sb2.reshape(1, D))
    outs = _group_ffn(bearr, bvarr, xs, ws, ew1, eb1, ew3, eb3, ew2, eb2)
    y = _combine(outs, z, p0, p1)
    return y
